# baseline (speedup 1.0000x reference)
import jax
import jax.numpy as jnp
from jax.experimental import pallas as pl

N = 50000
E = 800000
DIM = 128
NG = 128


def _gcn_conv(x, src, dst, W, b, n):
    h = x @ W
    loop = jnp.arange(n, dtype=src.dtype)
    s = jnp.concatenate([src, loop])
    d = jnp.concatenate([dst, loop])
    deg = jnp.zeros((n,), x.dtype).at[d].add(1.0)
    dis = jax.lax.rsqrt(deg)
    norm = dis[s] * dis[d]
    out = jnp.zeros((n, h.shape[1]), x.dtype).at[d].add(h[s] * norm[:, None])
    return out + b


def _gin_conv(x, src, dst, W1, b1, W2, b2, n):
    agg = jnp.zeros((n, x.shape[1]), x.dtype).at[dst].add(x[src])
    h = agg + x
    h = jax.nn.relu(h @ W1 + b1)
    return h @ W2 + b2


def _bn(x, gamma, beta, eps=1e-5):
    mean = jnp.mean(x, axis=0)
    var = jnp.var(x, axis=0)
    return (x - mean) * jax.lax.rsqrt(var + eps) * gamma + beta


def kernel(x, edge_index, batch, W_gcn1, b_gcn1, W_gcn2, b_gcn2, g0_W1, g0_b1, g0_W2, g0_b2, bn0_g, bn0_b, g1_W1, g1_b1, g1_W2, g1_b2, bn1_g, bn1_b, g2_W1, g2_b1, g2_W2, g2_b2, bn2_g, bn2_b):
    gin_params = [
        (g0_W1, g0_b1, g0_W2, g0_b2, bn0_g, bn0_b),
        (g1_W1, g1_b1, g1_W2, g1_b2, bn1_g, bn1_b),
        (g2_W1, g2_b1, g2_W2, g2_b2, bn2_g, bn2_b),
    ]
    n = x.shape[0]
    src = edge_index[0]
    dst = edge_index[1]
    x_gcn1 = jax.nn.relu(_gcn_conv(x, src, dst, W_gcn1, b_gcn1, n))
    x_gcn2_ = jax.nn.relu(_gcn_conv(x_gcn1, src, dst, W_gcn2, b_gcn2, n))
    x_gcn2_a = x_gcn2_ + x_gcn1
    x_gcn2_m = x_gcn2_ * x_gcn1
    h = x
    m = jnp.ones((n, DIM), x.dtype)
    e = jnp.zeros((n, DIM), x.dtype)
    lst = []
    for (W1, b1, W2, b2, g, bb) in gin_params:
        h = jax.nn.relu(_gin_conv(h, src, dst, W1, b1, W2, b2, n))
        h = _bn(h, g, bb)
        m = m * h
        e = e + h
        lst.append(h)
    node_rep = jnp.concatenate(lst + [m, e, x_gcn1, x_gcn2_, x_gcn2_a, x_gcn2_m], axis=-1)
    return jax.ops.segment_max(node_rep, batch, num_segments=NG)


# speedup vs baseline: 6.4570x; 6.4570x over previous
import functools

import jax
import jax.numpy as jnp
from jax import lax
from jax.experimental import pallas as pl
from jax.experimental.pallas import tpu as pltpu
from jax.experimental.pallas import tpu_sc as plsc

N = 50000
E = 800000
DIM = 128
NG = 128
NP = 50176
RPT = NP // 16
NBLK = E // 128
BPR = 392
FREL = 9 * DIM


@functools.partial(
    pl.kernel,
    out_type=(
        jax.ShapeDtypeStruct((NP,), jnp.float32),
        jax.ShapeDtypeStruct((144,), jnp.int32),
    ),
    mesh=plsc.VectorSubcoreMesh(core_axis_name="c", subcore_axis_name="s"),
    compiler_params=pltpu.CompilerParams(needs_layout_passes=False),
    scratch_types=[
        pltpu.VMEM((NP,), jnp.float32),
        pltpu.VMEM((2048,), jnp.float32),
        pltpu.VMEM((128,), jnp.float32),
        pltpu.VMEM((2, 2048), jnp.int32),
        pltpu.VMEM((RPT,), jnp.int32),
        pltpu.VMEM((128,), jnp.int32),
        pltpu.VMEM((2048,), jnp.int32),
        pltpu.VMEM((144,), jnp.int32),
        pltpu.VMEM_SHARED((16 * NP,), jnp.float32),
        pltpu.VMEM_SHARED((2048,), jnp.int32),
        pltpu.SemaphoreType.DMA,
    ],
)
def _sc_prep(ei_hbm, batch_hbm, deg_out, starts_out,
             histv, stage, outb, idxv, bstage, bhist, bstage16, startsv,
             sh_deg, sh_b, csem):
    c = lax.axis_index("c")
    s = lax.axis_index("s")
    zero16 = jnp.zeros((16,), jnp.float32)
    one16 = jnp.ones((16,), jnp.float32)
    izero16 = jnp.zeros((16,), jnp.int32)
    ione16 = jnp.ones((16,), jnp.int32)

    @pl.when(c == 0)
    def _():
        def zb(i, _):
            histv[pl.ds(i * 16, 16)] = zero16
            return 0
        lax.fori_loop(0, NP // 16, zb, 0)
        nbt2 = jnp.where(s < 7, 25, 24)

        def eb(b, _):
            bid = b * 16 + s
            nv = jnp.where(bid == 390, 80, 128)

            boff = pl.multiple_of(bid * 2048, 128)

            @pl.when(bid == 390)
            def _():
                pltpu.sync_copy(ei_hbm.at[:, pl.ds(boff, 1280)],
                                idxv.at[:, pl.ds(0, 1280)])

            @pl.when(bid < 390)
            def _():
                pltpu.sync_copy(ei_hbm.at[:, pl.ds(boff, 2048)], idxv)

            def inner(j, _):
                v = idxv[1, pl.ds(j * 16, 16)]
                plsc.addupdate_scatter(histv, [v], one16)
                return 0
            lax.fori_loop(0, nv, inner, 0)
            return 0
        lax.fori_loop(0, nbt2, eb, 0)
        pltpu.sync_copy(histv, sh_deg.at[pl.ds(s * NP, NP)])
        plsc.subcore_barrier()
        ncb = jnp.where(s < 8, 25, 24)

        def comb(b, _):
            bid = b * 16 + s
            for r in range(16):
                so = pl.multiple_of(r * NP + bid * 128, 8)
                pltpu.async_copy(sh_deg.at[pl.ds(so, 128)],
                                 stage.at[pl.ds(r * 128, 128)], csem)
            for r in range(16):
                so = pl.multiple_of(r * NP + bid * 128, 8)
                pltpu.make_async_copy(
                    sh_deg.at[pl.ds(so, 128)],
                    stage.at[pl.ds(r * 128, 128)], csem).wait()
            for j in range(8):
                t = jnp.full((16,), 1.0, jnp.float32)
                for r in range(16):
                    t = t + stage[pl.ds(r * 128 + j * 16, 16)]
                outb[pl.ds(j * 16, 16)] = t
            pltpu.sync_copy(
                outb, deg_out.at[pl.ds(pl.multiple_of(bid * 128, 8), 128)])
            return 0
        lax.fori_loop(0, ncb, comb, 0)

    @pl.when(c == 1)
    def _():
        for j in range(8):
            bhist[pl.ds(j * 16, 16)] = izero16
        pltpu.sync_copy(batch_hbm.at[pl.ds(s * RPT, RPT)], bstage)
        def inner(j, _):
            v = bstage[pl.ds(j * 16, 16)]
            plsc.addupdate_scatter(bhist, [v], ione16)
            return 0
        lax.fori_loop(0, RPT // 16, inner, 0)
        pltpu.sync_copy(bhist, sh_b.at[pl.ds(s * 128, 128)])
        plsc.subcore_barrier()

        @pl.when(s == 0)
        def _():
            pltpu.sync_copy(sh_b, bstage16)
            carry = jnp.int32(0)
            for j in range(8):
                t = bstage16[pl.ds(j * 16, 16)]
                for r in range(1, 16):
                    t = t + bstage16[pl.ds(r * 128 + j * 16, 16)]
                incl = plsc.cumsum(t)
                startsv[pl.ds(j * 16, 16)] = incl - t + carry
                carry = carry + jnp.sum(t, axis=0)
            startsv[pl.ds(128, 16)] = jnp.full((16,), N, jnp.int32)
            pltpu.sync_copy(startsv, starts_out)


def _make_spmm(n_chunks):
    cpc = n_chunks // 2

    @functools.partial(
        pl.kernel,
        out_type=jax.ShapeDtypeStruct((n_chunks * NP, 32), jnp.float32),
        mesh=plsc.VectorSubcoreMesh(core_axis_name="c", subcore_axis_name="s"),
        compiler_params=pltpu.CompilerParams(use_tc_tiling_on_sc=False),
        scratch_types=[
            pltpu.VMEM_SHARED((NP, 32), jnp.float32),
            pltpu.VMEM((2, 2, 128), jnp.int32),
            pltpu.VMEM((2, 128), jnp.int32),
            pltpu.VMEM((2, 128, 32), jnp.float32),
            pltpu.VMEM((392, 32), jnp.float32),
            pltpu.SemaphoreType.DMA,
            pltpu.SemaphoreType.DMA,
        ],
    )
    def k(y_hbm, ei_hbm, out_hbm, acc, ibuf, gbuf, rows, zv, sem0, sem1):
        c = lax.axis_index("c")
        s = lax.axis_index("s")
        sems = [sem0, sem1]
        zero16 = jnp.zeros((16,), jnp.float32)

        def zfill(i, _):
            zv[i // 2, pl.ds((i % 2) * 16, 16)] = zero16
            return 0
        lax.fori_loop(0, 392 * 2, zfill, 0)

        nbt = jnp.where(s < 10, NBLK // 16 + 1, NBLK // 16)

        def fetch(g, p, choff):
            off = pl.multiple_of((g * 16 + s) * 128, 128)
            pltpu.sync_copy(ei_hbm.at[:, pl.ds(off, 128)], ibuf.at[p])
            for j in range(8):
                gbuf[p, pl.ds(j * 16, 16)] = (
                    ibuf[p, 0, pl.ds(j * 16, 16)] + choff)
            pltpu.async_copy(y_hbm.at[gbuf.at[p]], rows.at[p], sems[p])

        def scat(p):
            pltpu.make_async_copy(y_hbm.at[gbuf.at[p]], rows.at[p],
                                  sems[p]).wait()
            pltpu.sync_copy(rows.at[p], acc.at[ibuf.at[p, 1]], add=True)

        for ci in range(cpc):
            choff = (c * cpc + ci) * NP
            for q in range(8):
                pltpu.sync_copy(zv, acc.at[pl.ds(s * RPT + q * 392, 392), :])
            plsc.subcore_barrier()

            fetch(0, 0, choff)

            def lbody(g, _):
                @pl.when(g % 2 == 0)
                def _():
                    fetch(g + 1, 1, choff)
                    scat(0)

                @pl.when(g % 2 == 1)
                def _():
                    fetch(g + 1, 0, choff)
                    scat(1)
                return 0
            lax.fori_loop(0, nbt - 1, lbody, 0)

            @pl.when((nbt - 1) % 2 == 0)
            def _():
                scat(0)

            @pl.when((nbt - 1) % 2 == 1)
            def _():
                scat(1)
            plsc.subcore_barrier()
            pltpu.sync_copy(acc.at[pl.ds(s * RPT, RPT), :],
                            out_hbm.at[pl.ds(choff + s * RPT, RPT), :])
    return k


_spmm6 = _make_spmm(6)
_spmm8 = _make_spmm(8)
_spmm4 = _make_spmm(4)


@functools.partial(
    pl.kernel,
    out_type=jax.ShapeDtypeStruct((NG * FREL,), jnp.float32),
    mesh=plsc.VectorSubcoreMesh(core_axis_name="c", subcore_axis_name="s"),
    scratch_types=[
        pltpu.VMEM((144,), jnp.int32),
        pltpu.VMEM((56, 640), jnp.float32),
        pltpu.VMEM((FREL,), jnp.float32),
    ],
)
def _sc_segmax(f_hbm, starts_hbm, out_hbm, startsv, rowbuf, accv):
    c = lax.axis_index("c")
    s = lax.axis_index("s")
    w = s * 2 + c
    pltpu.sync_copy(starts_hbm, startsv)
    ninf = jnp.full((16,), -jnp.inf, jnp.float32)

    for k in range(4):
        g = w * 4 + k
        r0 = startsv[pl.ds(g, 16)][0]
        r1 = startsv[pl.ds(g + 1, 16)][0]

        def zf(i, _):
            accv[pl.ds(i * 16, 16)] = ninf
            return 0
        lax.fori_loop(0, FREL // 16, zf, 0)

        nblk = (r1 - r0 + 47) // 48

        def blk(b, _):
            rs = r0 + b * 48
            rsa = pl.multiple_of((rs // 8) * 8, 8)
            off = rs - rsa
            cnt = jnp.minimum(48, r1 - rs)
            pltpu.sync_copy(f_hbm.at[pl.ds(rsa, 56), :], rowbuf)

            def row(i, _):
                ii = off + i
                for cc in range(8):
                    h0 = rowbuf[ii, pl.ds(cc * 16, 16)]
                    h1 = rowbuf[ii, pl.ds(128 + cc * 16, 16)]
                    h2 = rowbuf[ii, pl.ds(256 + cc * 16, 16)]
                    x1 = rowbuf[ii, pl.ds(384 + cc * 16, 16)]
                    x2 = rowbuf[ii, pl.ds(512 + cc * 16, 16)]
                    parts = (h0, h1, h2, h0 * h1 * h2, h0 + h1 + h2,
                             x1, x2, x2 + x1, x2 * x1)
                    for q in range(9):
                        sl = pl.ds(q * 128 + cc * 16, 16)
                        accv[sl] = jnp.maximum(accv[sl], parts[q])
                return 0
            lax.fori_loop(0, cnt, row, 0)
            return 0
        lax.fori_loop(0, nblk, blk, 0)
        pltpu.sync_copy(accv,
                        out_hbm.at[pl.ds(pl.multiple_of(g * FREL, 8), FREL)])


def _rowspec(w):
    return pl.BlockSpec((BPR, w), lambda i: (i, 0))


def _fullspec(shape):
    nd = len(shape)
    return pl.BlockSpec(shape, lambda i: (0,) * nd)


def _tca(xp, deg_col):
    def body(x_ref, d_ref, ya_ref):
        xb = x_ref[...]
        dis = lax.rsqrt(d_ref[...])
        u = dis * xb
        ya_ref[...] = jnp.stack(
            [u[:, 0:32], u[:, 32:64], u[:, 64:96],
             xb[:, 0:32], xb[:, 32:64], xb[:, 64:96]], axis=0)

    return pl.pallas_call(
        body, grid=(NP // BPR,),
        in_specs=[_rowspec(96), _rowspec(1)],
        out_specs=pl.BlockSpec((6, BPR, 32), lambda i: (0, i, 0)),
        out_shape=jax.ShapeDtypeStruct((6, NP, 32), jnp.float32),
    )(xp, deg_col)


def _sums_update(i, r, sums_ref):
    rid = i * BPR + lax.broadcasted_iota(jnp.int32, (BPR, 1), 0)
    rm = jnp.where(rid < N, r, 0.0)
    part = jnp.concatenate(
        [jnp.sum(rm, 0, keepdims=True), jnp.sum(rm * rm, 0, keepdims=True),
         jnp.zeros((6, DIM), jnp.float32)], axis=0)

    @pl.when(i == 0)
    def _():
        sums_ref[...] = part

    @pl.when(i > 0)
    def _():
        sums_ref[...] = sums_ref[...] + part


def _tcb(za, xp, deg_col, w1p, b1, gw1p, gb1, gw2, gb2):
    def body(za_ref, x_ref, d_ref, w1_ref, b1_ref, gw1_ref, gb1_ref,
             gw2_ref, gb2_ref, xg1_ref, r0_ref, sums_ref):
        i = pl.program_id(0)
        za_ = za_ref[...]
        xb = x_ref[...]
        dis = lax.rsqrt(d_ref[...])
        v1 = jnp.concatenate([za_[0], za_[1], za_[2]], axis=1)
        t1 = v1 + dis * xb
        xg1 = jnp.maximum(
            dis * jnp.dot(t1, w1_ref[...],
                          preferred_element_type=jnp.float32) + b1_ref[...],
            0.0)
        g0 = jnp.concatenate([za_[3], za_[4], za_[5]], axis=1) + xb
        t = jnp.maximum(
            jnp.dot(g0, gw1_ref[...], preferred_element_type=jnp.float32)
            + gb1_ref[...], 0.0)
        r0 = jnp.maximum(
            jnp.dot(t, gw2_ref[...], preferred_element_type=jnp.float32)
            + gb2_ref[...], 0.0)
        xg1_ref[...] = xg1
        r0_ref[...] = r0
        _sums_update(i, r0, sums_ref)

    return pl.pallas_call(
        body, grid=(NP // BPR,),
        in_specs=[pl.BlockSpec((6, BPR, 32), lambda i: (0, i, 0)),
                  _rowspec(96), _rowspec(1),
                  _fullspec((96, DIM)), _fullspec((1, DIM)),
                  _fullspec((96, DIM)), _fullspec((1, DIM)),
                  _fullspec((DIM, DIM)), _fullspec((1, DIM))],
        out_specs=[_rowspec(DIM), _rowspec(DIM),
                   pl.BlockSpec((8, DIM), lambda i: (0, 0))],
        out_shape=[jax.ShapeDtypeStruct((NP, DIM), jnp.float32),
                   jax.ShapeDtypeStruct((NP, DIM), jnp.float32),
                   jax.ShapeDtypeStruct((8, DIM), jnp.float32)],
    )(za, xp, deg_col, w1p, b1, gw1p, gb1, gw2, gb2)


def _bn_coefs(sums_ref, g_ref, b_ref):
    sm = sums_ref[...]
    mean = sm[0:1] / N
    var = sm[1:2] / N - mean * mean
    sc = g_ref[...] * lax.rsqrt(var + 1e-5)
    sh = b_ref[...] - mean * sc
    return sc, sh


def _tcc(r0, sums0, xg1, deg_col, bng, bnb):
    def body(r0_ref, sums_ref, xg1_ref, d_ref, g_ref, b_ref,
             yb_ref, h0_ref):
        sc, sh = _bn_coefs(sums_ref, g_ref, b_ref)
        h0 = r0_ref[...] * sc + sh
        dis = lax.rsqrt(d_ref[...])
        u2 = dis * xg1_ref[...]
        yb_ref[...] = jnp.stack(
            [u2[:, 0:32], u2[:, 32:64], u2[:, 64:96], u2[:, 96:128],
             h0[:, 0:32], h0[:, 32:64], h0[:, 64:96], h0[:, 96:128]], axis=0)
        h0_ref[...] = h0

    return pl.pallas_call(
        body, grid=(NP // BPR,),
        in_specs=[_rowspec(DIM), _fullspec((8, DIM)), _rowspec(DIM),
                  _rowspec(1), _fullspec((1, DIM)), _fullspec((1, DIM))],
        out_specs=[pl.BlockSpec((8, BPR, 32), lambda i: (0, i, 0)),
                   _rowspec(DIM)],
        out_shape=[jax.ShapeDtypeStruct((8, NP, 32), jnp.float32),
                   jax.ShapeDtypeStruct((NP, DIM), jnp.float32)],
    )(r0, sums0, xg1, deg_col, bng, bnb)


def _tcd(zb, xg1, h0, deg_col, w2, b2, gw1, gb1, gw2, gb2):
    def body(zb_ref, xg1_ref, h0_ref, d_ref, w2_ref, b2_ref, gw1_ref,
             gb1_ref, gw2_ref, gb2_ref, xg2_ref, r1_ref, sums_ref):
        i = pl.program_id(0)
        zb_ = zb_ref[...]
        au2 = jnp.concatenate([zb_[0], zb_[1], zb_[2], zb_[3]], axis=1)
        dis = lax.rsqrt(d_ref[...])
        xg1_ = xg1_ref[...]
        t2 = au2 + dis * xg1_
        xg2 = jnp.maximum(
            dis * jnp.dot(t2, w2_ref[...],
                          preferred_element_type=jnp.float32) + b2_ref[...],
            0.0)
        ah0 = jnp.concatenate([zb_[4], zb_[5], zb_[6], zb_[7]], axis=1)
        g1 = ah0 + h0_ref[...]
        t = jnp.maximum(
            jnp.dot(g1, gw1_ref[...], preferred_element_type=jnp.float32)
            + gb1_ref[...], 0.0)
        r1 = jnp.maximum(
            jnp.dot(t, gw2_ref[...], preferred_element_type=jnp.float32)
            + gb2_ref[...], 0.0)
        xg2_ref[...] = xg2
        r1_ref[...] = r1
        _sums_update(i, r1, sums_ref)

    return pl.pallas_call(
        body, grid=(NP // BPR,),
        in_specs=[pl.BlockSpec((8, BPR, 32), lambda i: (0, i, 0)),
                  _rowspec(DIM), _rowspec(DIM), _rowspec(1),
                  _fullspec((DIM, DIM)), _fullspec((1, DIM)),
                  _fullspec((DIM, DIM)), _fullspec((1, DIM)),
                  _fullspec((DIM, DIM)), _fullspec((1, DIM))],
        out_specs=[_rowspec(DIM), _rowspec(DIM),
                   pl.BlockSpec((8, DIM), lambda i: (0, 0))],
        out_shape=[jax.ShapeDtypeStruct((NP, DIM), jnp.float32),
                   jax.ShapeDtypeStruct((NP, DIM), jnp.float32),
                   jax.ShapeDtypeStruct((8, DIM), jnp.float32)],
    )(zb, xg1, h0, deg_col, w2, b2, gw1, gb1, gw2, gb2)


def _tce(r1, sums1, bng, bnb):
    def body(r1_ref, sums_ref, g_ref, b_ref, yc_ref, h1_ref):
        sc, sh = _bn_coefs(sums_ref, g_ref, b_ref)
        h1 = r1_ref[...] * sc + sh
        yc_ref[...] = jnp.stack(
            [h1[:, 0:32], h1[:, 32:64], h1[:, 64:96], h1[:, 96:128]], axis=0)
        h1_ref[...] = h1

    return pl.pallas_call(
        body, grid=(NP // BPR,),
        in_specs=[_rowspec(DIM), _fullspec((8, DIM)),
                  _fullspec((1, DIM)), _fullspec((1, DIM))],
        out_specs=[pl.BlockSpec((4, BPR, 32), lambda i: (0, i, 0)),
                   _rowspec(DIM)],
        out_shape=[jax.ShapeDtypeStruct((4, NP, 32), jnp.float32),
                   jax.ShapeDtypeStruct((NP, DIM), jnp.float32)],
    )(r1, sums1, bng, bnb)


def _tcf(zc, h1, gw1, gb1, gw2, gb2):
    def body(zc_ref, h1_ref, gw1_ref, gb1_ref, gw2_ref, gb2_ref,
             r2_ref, sums_ref):
        i = pl.program_id(0)
        zc_ = zc_ref[...]
        g2 = jnp.concatenate([zc_[0], zc_[1], zc_[2], zc_[3]], axis=1) \
            + h1_ref[...]
        t = jnp.maximum(
            jnp.dot(g2, gw1_ref[...], preferred_element_type=jnp.float32)
            + gb1_ref[...], 0.0)
        r2 = jnp.maximum(
            jnp.dot(t, gw2_ref[...], preferred_element_type=jnp.float32)
            + gb2_ref[...], 0.0)
        r2_ref[...] = r2
        _sums_update(i, r2, sums_ref)

    return pl.pallas_call(
        body, grid=(NP // BPR,),
        in_specs=[pl.BlockSpec((4, BPR, 32), lambda i: (0, i, 0)),
                  _rowspec(DIM),
                  _fullspec((DIM, DIM)), _fullspec((1, DIM)),
                  _fullspec((DIM, DIM)), _fullspec((1, DIM))],
        out_specs=[_rowspec(DIM), pl.BlockSpec((8, DIM), lambda i: (0, 0))],
        out_shape=[jax.ShapeDtypeStruct((NP, DIM), jnp.float32),
                   jax.ShapeDtypeStruct((8, DIM), jnp.float32)],
    )(zc, h1, gw1, gb1, gw2, gb2)


def _tcg(r2, sums2, h0, h1, xg1, xg2, bng, bnb):
    def body(r2_ref, sums_ref, g_ref, b_ref, h0_ref, h1_ref, xg1_ref,
             xg2_ref, f_ref):
        sc, sh = _bn_coefs(sums_ref, g_ref, b_ref)
        h2 = r2_ref[...] * sc + sh
        f_ref[...] = jnp.concatenate(
            [h0_ref[...], h1_ref[...], h2, xg1_ref[...], xg2_ref[...]],
            axis=1)

    return pl.pallas_call(
        body, grid=(NP // BPR,),
        in_specs=[_rowspec(DIM), _fullspec((8, DIM)),
                  _fullspec((1, DIM)), _fullspec((1, DIM)),
                  _rowspec(DIM), _rowspec(DIM), _rowspec(DIM), _rowspec(DIM)],
        out_specs=_rowspec(640),
        out_shape=jax.ShapeDtypeStruct((NP, 640), jnp.float32),
    )(r2, sums2, bng, bnb, h0, h1, xg1, xg2)


def kernel(x, edge_index, batch, W_gcn1, b_gcn1, W_gcn2, b_gcn2,
           g0_W1, g0_b1, g0_W2, g0_b2, bn0_g, bn0_b,
           g1_W1, g1_b1, g1_W2, g1_b2, bn1_g, bn1_b,
           g2_W1, g2_b1, g2_W2, g2_b2, bn2_g, bn2_b):
    row = lambda v: v.reshape(1, DIM)
    padw = lambda w: jnp.pad(w, ((0, 96 - w.shape[0]), (0, 0)))

    xp = jnp.pad(x, ((0, NP - N), (0, 96 - 78)))
    batch_pad = jnp.pad(batch, (0, NP - N), constant_values=NG - 1)

    deg, starts = _sc_prep(edge_index, batch_pad)
    deg_col = deg.reshape(NP, 1)

    ya = _tca(xp, deg_col)
    za = _spmm6(ya.reshape(6 * NP, 32), edge_index).reshape(6, NP, 32)
    xg1, r0, sums0 = _tcb(za, xp, deg_col, padw(W_gcn1), row(b_gcn1),
                          padw(g0_W1), row(g0_b1), g0_W2, row(g0_b2))
    yb, h0 = _tcc(r0, sums0, xg1, deg_col, row(bn0_g), row(bn0_b))
    zb = _spmm8(yb.reshape(8 * NP, 32), edge_index).reshape(8, NP, 32)
    xg2, r1, sums1 = _tcd(zb, xg1, h0, deg_col, W_gcn2, row(b_gcn2),
                          g1_W1, row(g1_b1), g1_W2, row(g1_b2))
    yc, h1 = _tce(r1, sums1, row(bn1_g), row(bn1_b))
    zc = _spmm4(yc.reshape(4 * NP, 32), edge_index).reshape(4, NP, 32)
    r2, sums2 = _tcf(zc, h1, g2_W1, row(g2_b1), g2_W2, row(g2_b2))
    f = _tcg(r2, sums2, h0, h1, xg1, xg2, row(bn2_g), row(bn2_b))
    out = _sc_segmax(f, starts)
    return out.reshape(NG, FREL)


# speedup vs baseline: 7.9325x; 1.2285x over previous
import functools

import jax
import jax.numpy as jnp
from jax import lax
from jax.experimental import pallas as pl
from jax.experimental.pallas import tpu as pltpu
from jax.experimental.pallas import tpu_sc as plsc

N = 50000
E = 800000
DIM = 128
NG = 128
NP = 50176
RPT = NP // 16
NBLK = E // 128
BPR = 392
FREL = 9 * DIM


@functools.partial(
    pl.kernel,
    out_type=(
        jax.ShapeDtypeStruct((NP,), jnp.float32),
        jax.ShapeDtypeStruct((144,), jnp.int32),
    ),
    mesh=plsc.VectorSubcoreMesh(core_axis_name="c", subcore_axis_name="s"),
    compiler_params=pltpu.CompilerParams(needs_layout_passes=False),
    scratch_types=[
        pltpu.VMEM((NP,), jnp.float32),
        pltpu.VMEM((2048,), jnp.float32),
        pltpu.VMEM((128,), jnp.float32),
        pltpu.VMEM((2, 2048), jnp.int32),
        pltpu.VMEM((RPT,), jnp.int32),
        pltpu.VMEM((128,), jnp.int32),
        pltpu.VMEM((2048,), jnp.int32),
        pltpu.VMEM((144,), jnp.int32),
        pltpu.VMEM_SHARED((16 * NP,), jnp.float32),
        pltpu.VMEM_SHARED((2048,), jnp.int32),
        pltpu.SemaphoreType.DMA,
    ],
)
def _sc_prep(ei_hbm, batch_hbm, deg_out, starts_out,
             histv, stage, outb, idxv, bstage, bhist, bstage16, startsv,
             sh_deg, sh_b, csem):
    c = lax.axis_index("c")
    s = lax.axis_index("s")
    zero16 = jnp.zeros((16,), jnp.float32)
    one16 = jnp.ones((16,), jnp.float32)
    izero16 = jnp.zeros((16,), jnp.int32)
    ione16 = jnp.ones((16,), jnp.int32)

    @pl.when(c == 0)
    def _():
        def zb(i, _):
            histv[pl.ds(i * 16, 16)] = zero16
            return 0
        lax.fori_loop(0, NP // 16, zb, 0)
        nbt2 = jnp.where(s < 7, 25, 24)

        def eb(b, _):
            bid = b * 16 + s
            nv = jnp.where(bid == 390, 80, 128)

            boff = pl.multiple_of(bid * 2048, 128)

            @pl.when(bid == 390)
            def _():
                pltpu.sync_copy(ei_hbm.at[:, pl.ds(boff, 1280)],
                                idxv.at[:, pl.ds(0, 1280)])

            @pl.when(bid < 390)
            def _():
                pltpu.sync_copy(ei_hbm.at[:, pl.ds(boff, 2048)], idxv)

            def inner(j, _):
                v = idxv[1, pl.ds(j * 16, 16)]
                plsc.addupdate_scatter(histv, [v], one16)
                return 0
            lax.fori_loop(0, nv, inner, 0)
            return 0
        lax.fori_loop(0, nbt2, eb, 0)
        pltpu.sync_copy(histv, sh_deg.at[pl.ds(s * NP, NP)])
        plsc.subcore_barrier()
        ncb = jnp.where(s < 8, 25, 24)

        def comb(b, _):
            bid = b * 16 + s
            for r in range(16):
                so = pl.multiple_of(r * NP + bid * 128, 8)
                pltpu.async_copy(sh_deg.at[pl.ds(so, 128)],
                                 stage.at[pl.ds(r * 128, 128)], csem)
            for r in range(16):
                so = pl.multiple_of(r * NP + bid * 128, 8)
                pltpu.make_async_copy(
                    sh_deg.at[pl.ds(so, 128)],
                    stage.at[pl.ds(r * 128, 128)], csem).wait()
            for j in range(8):
                t = jnp.full((16,), 1.0, jnp.float32)
                for r in range(16):
                    t = t + stage[pl.ds(r * 128 + j * 16, 16)]
                outb[pl.ds(j * 16, 16)] = t
            pltpu.sync_copy(
                outb, deg_out.at[pl.ds(pl.multiple_of(bid * 128, 8), 128)])
            return 0
        lax.fori_loop(0, ncb, comb, 0)

    @pl.when(c == 1)
    def _():
        for j in range(8):
            bhist[pl.ds(j * 16, 16)] = izero16
        pltpu.sync_copy(batch_hbm.at[pl.ds(s * RPT, RPT)], bstage)
        def inner(j, _):
            v = bstage[pl.ds(j * 16, 16)]
            plsc.addupdate_scatter(bhist, [v], ione16)
            return 0
        lax.fori_loop(0, RPT // 16, inner, 0)
        pltpu.sync_copy(bhist, sh_b.at[pl.ds(s * 128, 128)])
        plsc.subcore_barrier()

        @pl.when(s == 0)
        def _():
            pltpu.sync_copy(sh_b, bstage16)
            carry = jnp.int32(0)
            for j in range(8):
                t = bstage16[pl.ds(j * 16, 16)]
                for r in range(1, 16):
                    t = t + bstage16[pl.ds(r * 128 + j * 16, 16)]
                incl = plsc.cumsum(t)
                startsv[pl.ds(j * 16, 16)] = incl - t + carry
                carry = carry + jnp.sum(t, axis=0)
            startsv[pl.ds(128, 16)] = jnp.full((16,), N, jnp.int32)
            pltpu.sync_copy(startsv, starts_out)


NBE = 391
E_PAD = 16 * NBE * 128


def _make_spmm(n_chunks):
    cpc = n_chunks // 2

    @functools.partial(
        pl.kernel,
        out_type=jax.ShapeDtypeStruct((n_chunks * NP, 32), jnp.float32),
        mesh=plsc.VectorSubcoreMesh(core_axis_name="c", subcore_axis_name="s"),
        compiler_params=pltpu.CompilerParams(use_tc_tiling_on_sc=False),
        scratch_types=[
            pltpu.VMEM_SHARED((NP, 32), jnp.float32),
            pltpu.VMEM((4, 2, 128), jnp.int32),
            pltpu.VMEM((2, 128), jnp.int32),
            pltpu.VMEM((2, 128, 32), jnp.float32),
            pltpu.VMEM((392, 32), jnp.float32),
            pltpu.SemaphoreType.DMA, pltpu.SemaphoreType.DMA,
            pltpu.SemaphoreType.DMA, pltpu.SemaphoreType.DMA,
            pltpu.SemaphoreType.DMA, pltpu.SemaphoreType.DMA,
            pltpu.SemaphoreType.DMA, pltpu.SemaphoreType.DMA,
        ],
    )
    def k(y_hbm, ei_hbm, out_hbm, acc, ibuf, gbuf, rows, zv,
          is0, is1, is2, is3, gs0, gs1, ss0, ss1):
        c = lax.axis_index("c")
        s = lax.axis_index("s")
        isems = [is0, is1, is2, is3]
        gsems = [gs0, gs1]
        ssems = [ss0, ss1]
        zero16 = jnp.zeros((16,), jnp.float32)
        base = s * NBE

        def zfill(i, _):
            zv[i // 2, pl.ds((i % 2) * 16, 16)] = zero16
            return 0
        lax.fori_loop(0, 392 * 2, zfill, 0)

        def eslice(g):
            off = pl.multiple_of((base + g) * 128, 128)
            return ei_hbm.at[:, pl.ds(off, 128)]

        def idx_start(g, k4):
            pltpu.async_copy(eslice(g), ibuf.at[k4], isems[k4])

        def idx_wait(g, k4):
            pltpu.make_async_copy(eslice(g), ibuf.at[k4], isems[k4]).wait()

        def gat_start(k4, k2, choff):
            for j in range(8):
                gbuf[k2, pl.ds(j * 16, 16)] = (
                    ibuf[k4, 0, pl.ds(j * 16, 16)] + choff)
            pltpu.async_copy(y_hbm.at[gbuf.at[k2]], rows.at[k2], gsems[k2])

        def gat_wait(k2):
            pltpu.make_async_copy(y_hbm.at[gbuf.at[k2]], rows.at[k2],
                                  gsems[k2]).wait()

        def sc_start(k4, k2):
            pltpu.async_copy(rows.at[k2], acc.at[ibuf.at[k4, 1]], ssems[k2],
                             add=True)

        def sc_wait(k4, k2):
            pltpu.make_async_copy(rows.at[k2], acc.at[ibuf.at[k4, 1]],
                                  ssems[k2]).wait()

        for ci in range(cpc):
            choff = (c * cpc + ci) * NP
            for q in range(8):
                pltpu.sync_copy(zv, acc.at[pl.ds(s * RPT + q * 392, 392), :])
            plsc.subcore_barrier()

            idx_start(0, 0)
            idx_start(1, 1)
            idx_wait(0, 0)
            gat_start(0, 0, choff)
            idx_start(2, 2)
            idx_wait(1, 1)
            gat_start(1, 1, choff)
            gat_wait(0)
            sc_start(0, 0)
            idx_start(3, 3)

            def full_iter(g, k4, k2):
                idx_wait(g, k4)
                sc_wait((k4 + 2) % 4, k2)
                gat_start(k4, k2, choff)
                gat_wait(1 - k2)
                sc_start((k4 + 3) % 4, 1 - k2)

            def lbody(i, _):
                for kk in range(4):
                    g = 2 + i * 4 + kk
                    full_iter(g, (2 + kk) % 4, kk % 2)
                    idx_start(g + 2, kk)
                return 0
            lax.fori_loop(0, 96, lbody, 0)

            for g in (386, 387, 388, 389, 390):
                full_iter(g, g % 4, g % 2)
                if g + 2 <= 390:
                    idx_start(g + 2, (g + 2) % 4)
            gat_wait(0)
            sc_start(2, 0)
            sc_wait(1, 1)
            sc_wait(2, 0)

            plsc.subcore_barrier()
            pltpu.sync_copy(acc.at[pl.ds(s * RPT, RPT), :],
                            out_hbm.at[pl.ds(choff + s * RPT, RPT), :])
    return k


_spmm6 = _make_spmm(6)
_spmm8 = _make_spmm(8)
_spmm4 = _make_spmm(4)


@functools.partial(
    pl.kernel,
    out_type=jax.ShapeDtypeStruct((NG * FREL,), jnp.float32),
    mesh=plsc.VectorSubcoreMesh(core_axis_name="c", subcore_axis_name="s"),
    scratch_types=[
        pltpu.VMEM((144,), jnp.int32),
        pltpu.VMEM((56, 640), jnp.float32),
        pltpu.VMEM((FREL,), jnp.float32),
    ],
)
def _sc_segmax(f_hbm, starts_hbm, out_hbm, startsv, rowbuf, accv):
    c = lax.axis_index("c")
    s = lax.axis_index("s")
    w = s * 2 + c
    pltpu.sync_copy(starts_hbm, startsv)
    ninf = jnp.full((16,), -jnp.inf, jnp.float32)

    for k in range(4):
        g = w * 4 + k
        r0 = startsv[pl.ds(g, 16)][0]
        r1 = startsv[pl.ds(g + 1, 16)][0]

        def zf(i, _):
            accv[pl.ds(i * 16, 16)] = ninf
            return 0
        lax.fori_loop(0, FREL // 16, zf, 0)

        nblk = (r1 - r0 + 47) // 48

        def blk(b, _):
            rs = r0 + b * 48
            rsa = pl.multiple_of((rs // 8) * 8, 8)
            off = rs - rsa
            cnt = jnp.minimum(48, r1 - rs)
            pltpu.sync_copy(f_hbm.at[pl.ds(rsa, 56), :], rowbuf)

            def row(i, _):
                ii = off + i
                for cc in range(8):
                    h0 = rowbuf[ii, pl.ds(cc * 16, 16)]
                    h1 = rowbuf[ii, pl.ds(128 + cc * 16, 16)]
                    h2 = rowbuf[ii, pl.ds(256 + cc * 16, 16)]
                    x1 = rowbuf[ii, pl.ds(384 + cc * 16, 16)]
                    x2 = rowbuf[ii, pl.ds(512 + cc * 16, 16)]
                    parts = (h0, h1, h2, h0 * h1 * h2, h0 + h1 + h2,
                             x1, x2, x2 + x1, x2 * x1)
                    for q in range(9):
                        sl = pl.ds(q * 128 + cc * 16, 16)
                        accv[sl] = jnp.maximum(accv[sl], parts[q])
                return 0
            lax.fori_loop(0, cnt, row, 0)
            return 0
        lax.fori_loop(0, nblk, blk, 0)
        pltpu.sync_copy(accv,
                        out_hbm.at[pl.ds(pl.multiple_of(g * FREL, 8), FREL)])


def _rowspec(w):
    return pl.BlockSpec((BPR, w), lambda i: (i, 0))


def _fullspec(shape):
    nd = len(shape)
    return pl.BlockSpec(shape, lambda i: (0,) * nd)


def _tca(xp, deg_col):
    def body(x_ref, d_ref, ya_ref):
        xb = x_ref[...]
        dis = lax.rsqrt(d_ref[...])
        u = dis * xb
        ya_ref[...] = jnp.stack(
            [u[:, 0:32], u[:, 32:64], u[:, 64:96],
             xb[:, 0:32], xb[:, 32:64], xb[:, 64:96]], axis=0)

    return pl.pallas_call(
        body, grid=(NP // BPR,),
        in_specs=[_rowspec(96), _rowspec(1)],
        out_specs=pl.BlockSpec((6, BPR, 32), lambda i: (0, i, 0)),
        out_shape=jax.ShapeDtypeStruct((6, NP, 32), jnp.float32),
    )(xp, deg_col)


def _sums_update(i, r, sums_ref):
    rid = i * BPR + lax.broadcasted_iota(jnp.int32, (BPR, 1), 0)
    rm = jnp.where(rid < N, r, 0.0)
    part = jnp.concatenate(
        [jnp.sum(rm, 0, keepdims=True), jnp.sum(rm * rm, 0, keepdims=True),
         jnp.zeros((6, DIM), jnp.float32)], axis=0)

    @pl.when(i == 0)
    def _():
        sums_ref[...] = part

    @pl.when(i > 0)
    def _():
        sums_ref[...] = sums_ref[...] + part


def _tcb(za, xp, deg_col, w1p, b1, gw1p, gb1, gw2, gb2):
    def body(za_ref, x_ref, d_ref, w1_ref, b1_ref, gw1_ref, gb1_ref,
             gw2_ref, gb2_ref, xg1_ref, r0_ref, sums_ref):
        i = pl.program_id(0)
        za_ = za_ref[...]
        xb = x_ref[...]
        dis = lax.rsqrt(d_ref[...])
        v1 = jnp.concatenate([za_[0], za_[1], za_[2]], axis=1)
        t1 = v1 + dis * xb
        xg1 = jnp.maximum(
            dis * jnp.dot(t1, w1_ref[...],
                          preferred_element_type=jnp.float32) + b1_ref[...],
            0.0)
        g0 = jnp.concatenate([za_[3], za_[4], za_[5]], axis=1) + xb
        t = jnp.maximum(
            jnp.dot(g0, gw1_ref[...], preferred_element_type=jnp.float32)
            + gb1_ref[...], 0.0)
        r0 = jnp.maximum(
            jnp.dot(t, gw2_ref[...], preferred_element_type=jnp.float32)
            + gb2_ref[...], 0.0)
        xg1_ref[...] = xg1
        r0_ref[...] = r0
        _sums_update(i, r0, sums_ref)

    return pl.pallas_call(
        body, grid=(NP // BPR,),
        in_specs=[pl.BlockSpec((6, BPR, 32), lambda i: (0, i, 0)),
                  _rowspec(96), _rowspec(1),
                  _fullspec((96, DIM)), _fullspec((1, DIM)),
                  _fullspec((96, DIM)), _fullspec((1, DIM)),
                  _fullspec((DIM, DIM)), _fullspec((1, DIM))],
        out_specs=[_rowspec(DIM), _rowspec(DIM),
                   pl.BlockSpec((8, DIM), lambda i: (0, 0))],
        out_shape=[jax.ShapeDtypeStruct((NP, DIM), jnp.float32),
                   jax.ShapeDtypeStruct((NP, DIM), jnp.float32),
                   jax.ShapeDtypeStruct((8, DIM), jnp.float32)],
    )(za, xp, deg_col, w1p, b1, gw1p, gb1, gw2, gb2)


def _bn_coefs(sums_ref, g_ref, b_ref):
    sm = sums_ref[...]
    mean = sm[0:1] / N
    var = sm[1:2] / N - mean * mean
    sc = g_ref[...] * lax.rsqrt(var + 1e-5)
    sh = b_ref[...] - mean * sc
    return sc, sh


def _tcc(r0, sums0, xg1, deg_col, bng, bnb):
    def body(r0_ref, sums_ref, xg1_ref, d_ref, g_ref, b_ref,
             yb_ref, h0_ref):
        sc, sh = _bn_coefs(sums_ref, g_ref, b_ref)
        h0 = r0_ref[...] * sc + sh
        dis = lax.rsqrt(d_ref[...])
        u2 = dis * xg1_ref[...]
        yb_ref[...] = jnp.stack(
            [u2[:, 0:32], u2[:, 32:64], u2[:, 64:96], u2[:, 96:128],
             h0[:, 0:32], h0[:, 32:64], h0[:, 64:96], h0[:, 96:128]], axis=0)
        h0_ref[...] = h0

    return pl.pallas_call(
        body, grid=(NP // BPR,),
        in_specs=[_rowspec(DIM), _fullspec((8, DIM)), _rowspec(DIM),
                  _rowspec(1), _fullspec((1, DIM)), _fullspec((1, DIM))],
        out_specs=[pl.BlockSpec((8, BPR, 32), lambda i: (0, i, 0)),
                   _rowspec(DIM)],
        out_shape=[jax.ShapeDtypeStruct((8, NP, 32), jnp.float32),
                   jax.ShapeDtypeStruct((NP, DIM), jnp.float32)],
    )(r0, sums0, xg1, deg_col, bng, bnb)


def _tcd(zb, xg1, h0, deg_col, w2, b2, gw1, gb1, gw2, gb2):
    def body(zb_ref, xg1_ref, h0_ref, d_ref, w2_ref, b2_ref, gw1_ref,
             gb1_ref, gw2_ref, gb2_ref, xg2_ref, r1_ref, sums_ref):
        i = pl.program_id(0)
        zb_ = zb_ref[...]
        au2 = jnp.concatenate([zb_[0], zb_[1], zb_[2], zb_[3]], axis=1)
        dis = lax.rsqrt(d_ref[...])
        xg1_ = xg1_ref[...]
        t2 = au2 + dis * xg1_
        xg2 = jnp.maximum(
            dis * jnp.dot(t2, w2_ref[...],
                          preferred_element_type=jnp.float32) + b2_ref[...],
            0.0)
        ah0 = jnp.concatenate([zb_[4], zb_[5], zb_[6], zb_[7]], axis=1)
        g1 = ah0 + h0_ref[...]
        t = jnp.maximum(
            jnp.dot(g1, gw1_ref[...], preferred_element_type=jnp.float32)
            + gb1_ref[...], 0.0)
        r1 = jnp.maximum(
            jnp.dot(t, gw2_ref[...], preferred_element_type=jnp.float32)
            + gb2_ref[...], 0.0)
        xg2_ref[...] = xg2
        r1_ref[...] = r1
        _sums_update(i, r1, sums_ref)

    return pl.pallas_call(
        body, grid=(NP // BPR,),
        in_specs=[pl.BlockSpec((8, BPR, 32), lambda i: (0, i, 0)),
                  _rowspec(DIM), _rowspec(DIM), _rowspec(1),
                  _fullspec((DIM, DIM)), _fullspec((1, DIM)),
                  _fullspec((DIM, DIM)), _fullspec((1, DIM)),
                  _fullspec((DIM, DIM)), _fullspec((1, DIM))],
        out_specs=[_rowspec(DIM), _rowspec(DIM),
                   pl.BlockSpec((8, DIM), lambda i: (0, 0))],
        out_shape=[jax.ShapeDtypeStruct((NP, DIM), jnp.float32),
                   jax.ShapeDtypeStruct((NP, DIM), jnp.float32),
                   jax.ShapeDtypeStruct((8, DIM), jnp.float32)],
    )(zb, xg1, h0, deg_col, w2, b2, gw1, gb1, gw2, gb2)


def _tce(r1, sums1, bng, bnb):
    def body(r1_ref, sums_ref, g_ref, b_ref, yc_ref, h1_ref):
        sc, sh = _bn_coefs(sums_ref, g_ref, b_ref)
        h1 = r1_ref[...] * sc + sh
        yc_ref[...] = jnp.stack(
            [h1[:, 0:32], h1[:, 32:64], h1[:, 64:96], h1[:, 96:128]], axis=0)
        h1_ref[...] = h1

    return pl.pallas_call(
        body, grid=(NP // BPR,),
        in_specs=[_rowspec(DIM), _fullspec((8, DIM)),
                  _fullspec((1, DIM)), _fullspec((1, DIM))],
        out_specs=[pl.BlockSpec((4, BPR, 32), lambda i: (0, i, 0)),
                   _rowspec(DIM)],
        out_shape=[jax.ShapeDtypeStruct((4, NP, 32), jnp.float32),
                   jax.ShapeDtypeStruct((NP, DIM), jnp.float32)],
    )(r1, sums1, bng, bnb)


def _tcf(zc, h1, gw1, gb1, gw2, gb2):
    def body(zc_ref, h1_ref, gw1_ref, gb1_ref, gw2_ref, gb2_ref,
             r2_ref, sums_ref):
        i = pl.program_id(0)
        zc_ = zc_ref[...]
        g2 = jnp.concatenate([zc_[0], zc_[1], zc_[2], zc_[3]], axis=1) \
            + h1_ref[...]
        t = jnp.maximum(
            jnp.dot(g2, gw1_ref[...], preferred_element_type=jnp.float32)
            + gb1_ref[...], 0.0)
        r2 = jnp.maximum(
            jnp.dot(t, gw2_ref[...], preferred_element_type=jnp.float32)
            + gb2_ref[...], 0.0)
        r2_ref[...] = r2
        _sums_update(i, r2, sums_ref)

    return pl.pallas_call(
        body, grid=(NP // BPR,),
        in_specs=[pl.BlockSpec((4, BPR, 32), lambda i: (0, i, 0)),
                  _rowspec(DIM),
                  _fullspec((DIM, DIM)), _fullspec((1, DIM)),
                  _fullspec((DIM, DIM)), _fullspec((1, DIM))],
        out_specs=[_rowspec(DIM), pl.BlockSpec((8, DIM), lambda i: (0, 0))],
        out_shape=[jax.ShapeDtypeStruct((NP, DIM), jnp.float32),
                   jax.ShapeDtypeStruct((8, DIM), jnp.float32)],
    )(zc, h1, gw1, gb1, gw2, gb2)


def _tcg(r2, sums2, h0, h1, xg1, xg2, bng, bnb):
    def body(r2_ref, sums_ref, g_ref, b_ref, h0_ref, h1_ref, xg1_ref,
             xg2_ref, f_ref):
        sc, sh = _bn_coefs(sums_ref, g_ref, b_ref)
        h2 = r2_ref[...] * sc + sh
        f_ref[...] = jnp.concatenate(
            [h0_ref[...], h1_ref[...], h2, xg1_ref[...], xg2_ref[...]],
            axis=1)

    return pl.pallas_call(
        body, grid=(NP // BPR,),
        in_specs=[_rowspec(DIM), _fullspec((8, DIM)),
                  _fullspec((1, DIM)), _fullspec((1, DIM)),
                  _rowspec(DIM), _rowspec(DIM), _rowspec(DIM), _rowspec(DIM)],
        out_specs=_rowspec(640),
        out_shape=jax.ShapeDtypeStruct((NP, 640), jnp.float32),
    )(r2, sums2, bng, bnb, h0, h1, xg1, xg2)


def kernel(x, edge_index, batch, W_gcn1, b_gcn1, W_gcn2, b_gcn2,
           g0_W1, g0_b1, g0_W2, g0_b2, bn0_g, bn0_b,
           g1_W1, g1_b1, g1_W2, g1_b2, bn1_g, bn1_b,
           g2_W1, g2_b1, g2_W2, g2_b2, bn2_g, bn2_b):
    row = lambda v: v.reshape(1, DIM)
    padw = lambda w: jnp.pad(w, ((0, 96 - w.shape[0]), (0, 0)))

    xp = jnp.pad(x, ((0, NP - N), (0, 96 - 78)))
    batch_pad = jnp.pad(batch, (0, NP - N), constant_values=NG - 1)
    ei_pad = jnp.pad(edge_index, ((0, 0), (0, E_PAD - E)),
                     constant_values=NP - 1)

    deg, starts = _sc_prep(ei_pad, batch_pad)
    deg_col = deg.reshape(NP, 1)

    ya = _tca(xp, deg_col)
    za = _spmm6(ya.reshape(6 * NP, 32), ei_pad).reshape(6, NP, 32)
    xg1, r0, sums0 = _tcb(za, xp, deg_col, padw(W_gcn1), row(b_gcn1),
                          padw(g0_W1), row(g0_b1), g0_W2, row(g0_b2))
    yb, h0 = _tcc(r0, sums0, xg1, deg_col, row(bn0_g), row(bn0_b))
    zb = _spmm8(yb.reshape(8 * NP, 32), ei_pad).reshape(8, NP, 32)
    xg2, r1, sums1 = _tcd(zb, xg1, h0, deg_col, W_gcn2, row(b_gcn2),
                          g1_W1, row(g1_b1), g1_W2, row(g1_b2))
    yc, h1 = _tce(r1, sums1, row(bn1_g), row(bn1_b))
    zc = _spmm4(yc.reshape(4 * NP, 32), ei_pad).reshape(4, NP, 32)
    r2, sums2 = _tcf(zc, h1, g2_W1, row(g2_b1), g2_W2, row(g2_b2))
    f = _tcg(r2, sums2, h0, h1, xg1, xg2, row(bn2_g), row(bn2_b))
    out = _sc_segmax(f, starts)
    return out.reshape(NG, FREL)


# speedup vs baseline: 8.0288x; 1.0121x over previous
import functools

import jax
import jax.numpy as jnp
from jax import lax
from jax.experimental import pallas as pl
from jax.experimental.pallas import tpu as pltpu
from jax.experimental.pallas import tpu_sc as plsc

N = 50000
E = 800000
DIM = 128
NG = 128
NP = 50176
RPT = NP // 16
NBLK = E // 128
BPR = 392
FREL = 9 * DIM


@functools.partial(
    pl.kernel,
    out_type=(
        jax.ShapeDtypeStruct((NP,), jnp.float32),
        jax.ShapeDtypeStruct((144,), jnp.int32),
    ),
    mesh=plsc.VectorSubcoreMesh(core_axis_name="c", subcore_axis_name="s"),
    compiler_params=pltpu.CompilerParams(needs_layout_passes=False),
    scratch_types=[
        pltpu.VMEM((NP,), jnp.float32),
        pltpu.VMEM((2048,), jnp.float32),
        pltpu.VMEM((128,), jnp.float32),
        pltpu.VMEM((2, 2048), jnp.int32),
        pltpu.VMEM((RPT,), jnp.int32),
        pltpu.VMEM((128,), jnp.int32),
        pltpu.VMEM((2048,), jnp.int32),
        pltpu.VMEM((144,), jnp.int32),
        pltpu.VMEM_SHARED((16 * NP,), jnp.float32),
        pltpu.VMEM_SHARED((2048,), jnp.int32),
        pltpu.SemaphoreType.DMA,
    ],
)
def _sc_prep(ei_hbm, batch_hbm, deg_out, starts_out,
             histv, stage, outb, idxv, bstage, bhist, bstage16, startsv,
             sh_deg, sh_b, csem):
    c = lax.axis_index("c")
    s = lax.axis_index("s")
    zero16 = jnp.zeros((16,), jnp.float32)
    one16 = jnp.ones((16,), jnp.float32)
    izero16 = jnp.zeros((16,), jnp.int32)
    ione16 = jnp.ones((16,), jnp.int32)

    @pl.when(c == 0)
    def _():
        def zb(i, _):
            histv[pl.ds(i * 16, 16)] = zero16
            return 0
        lax.fori_loop(0, NP // 16, zb, 0)
        nbt2 = jnp.where(s < 7, 25, 24)

        def eb(b, _):
            bid = b * 16 + s
            nv = jnp.where(bid == 390, 80, 128)

            boff = pl.multiple_of(bid * 2048, 128)

            @pl.when(bid == 390)
            def _():
                pltpu.sync_copy(ei_hbm.at[:, pl.ds(boff, 1280)],
                                idxv.at[:, pl.ds(0, 1280)])

            @pl.when(bid < 390)
            def _():
                pltpu.sync_copy(ei_hbm.at[:, pl.ds(boff, 2048)], idxv)

            def inner(j, _):
                v = idxv[1, pl.ds(j * 16, 16)]
                plsc.addupdate_scatter(histv, [v], one16)
                return 0
            lax.fori_loop(0, nv, inner, 0)
            return 0
        lax.fori_loop(0, nbt2, eb, 0)
        pltpu.sync_copy(histv, sh_deg.at[pl.ds(s * NP, NP)])
        plsc.subcore_barrier()
        ncb = jnp.where(s < 8, 25, 24)

        def comb(b, _):
            bid = b * 16 + s
            for r in range(16):
                so = pl.multiple_of(r * NP + bid * 128, 8)
                pltpu.async_copy(sh_deg.at[pl.ds(so, 128)],
                                 stage.at[pl.ds(r * 128, 128)], csem)
            for r in range(16):
                so = pl.multiple_of(r * NP + bid * 128, 8)
                pltpu.make_async_copy(
                    sh_deg.at[pl.ds(so, 128)],
                    stage.at[pl.ds(r * 128, 128)], csem).wait()
            for j in range(8):
                t = jnp.full((16,), 1.0, jnp.float32)
                for r in range(16):
                    t = t + stage[pl.ds(r * 128 + j * 16, 16)]
                outb[pl.ds(j * 16, 16)] = t
            pltpu.sync_copy(
                outb, deg_out.at[pl.ds(pl.multiple_of(bid * 128, 8), 128)])
            return 0
        lax.fori_loop(0, ncb, comb, 0)

    @pl.when(c == 1)
    def _():
        for j in range(8):
            bhist[pl.ds(j * 16, 16)] = izero16
        pltpu.sync_copy(batch_hbm.at[pl.ds(s * RPT, RPT)], bstage)
        def inner(j, _):
            v = bstage[pl.ds(j * 16, 16)]
            plsc.addupdate_scatter(bhist, [v], ione16)
            return 0
        lax.fori_loop(0, RPT // 16, inner, 0)
        pltpu.sync_copy(bhist, sh_b.at[pl.ds(s * 128, 128)])
        plsc.subcore_barrier()

        @pl.when(s == 0)
        def _():
            pltpu.sync_copy(sh_b, bstage16)
            carry = jnp.int32(0)
            for j in range(8):
                t = bstage16[pl.ds(j * 16, 16)]
                for r in range(1, 16):
                    t = t + bstage16[pl.ds(r * 128 + j * 16, 16)]
                incl = plsc.cumsum(t)
                startsv[pl.ds(j * 16, 16)] = incl - t + carry
                carry = carry + jnp.sum(t, axis=0)
            startsv[pl.ds(128, 16)] = jnp.full((16,), N, jnp.int32)
            pltpu.sync_copy(startsv, starts_out)


NBE = 391
E_PAD = 16 * NBE * 128


def _make_spmm(n_chunks):
    cpc = n_chunks // 2

    @functools.partial(
        pl.kernel,
        out_type=jax.ShapeDtypeStruct((n_chunks * NP, 32), jnp.float32),
        mesh=plsc.VectorSubcoreMesh(core_axis_name="c", subcore_axis_name="s"),
        compiler_params=pltpu.CompilerParams(use_tc_tiling_on_sc=False),
        scratch_types=[
            pltpu.VMEM_SHARED((NP, 32), jnp.float32),
            pltpu.VMEM((4, 2, 128), jnp.int32),
            pltpu.VMEM((2, 128), jnp.int32),
            pltpu.VMEM((2, 128, 32), jnp.float32),
            pltpu.VMEM((392, 32), jnp.float32),
            pltpu.SemaphoreType.DMA, pltpu.SemaphoreType.DMA,
            pltpu.SemaphoreType.DMA, pltpu.SemaphoreType.DMA,
            pltpu.SemaphoreType.DMA, pltpu.SemaphoreType.DMA,
            pltpu.SemaphoreType.DMA, pltpu.SemaphoreType.DMA,
        ],
    )
    def k(y_hbm, ei_hbm, out_hbm, acc, ibuf, gbuf, rows, zv,
          is0, is1, is2, is3, gs0, gs1, ss0, ss1):
        c = lax.axis_index("c")
        s = lax.axis_index("s")
        isems = [is0, is1, is2, is3]
        gsems = [gs0, gs1]
        ssems = [ss0, ss1]
        zero16 = jnp.zeros((16,), jnp.float32)
        base = s * NBE

        def zfill(i, _):
            zv[i // 2, pl.ds((i % 2) * 16, 16)] = zero16
            return 0
        lax.fori_loop(0, 392 * 2, zfill, 0)

        def eslice(g):
            off = pl.multiple_of((base + g) * 128, 128)
            return ei_hbm.at[:, pl.ds(off, 128)]

        def idx_start(g, k4):
            pltpu.async_copy(eslice(g), ibuf.at[k4], isems[k4])

        def idx_wait(g, k4):
            pltpu.make_async_copy(eslice(g), ibuf.at[k4], isems[k4]).wait()

        def gat_start(k4, k2, choff):
            for j in range(8):
                gbuf[k2, pl.ds(j * 16, 16)] = (
                    ibuf[k4, 0, pl.ds(j * 16, 16)] + choff)
            pltpu.async_copy(y_hbm.at[gbuf.at[k2]], rows.at[k2], gsems[k2])

        def gat_wait(k2):
            pltpu.make_async_copy(y_hbm.at[gbuf.at[k2]], rows.at[k2],
                                  gsems[k2]).wait()

        def sc_start(k4, k2):
            pltpu.async_copy(rows.at[k2], acc.at[ibuf.at[k4, 1]], ssems[k2],
                             add=True)

        def sc_wait(k4, k2):
            pltpu.make_async_copy(rows.at[k2], acc.at[ibuf.at[k4, 1]],
                                  ssems[k2]).wait()

        for ci in range(cpc):
            choff = (c * cpc + ci) * NP
            for q in range(8):
                pltpu.sync_copy(zv, acc.at[pl.ds(s * RPT + q * 392, 392), :])
            plsc.subcore_barrier()

            idx_start(0, 0)
            idx_start(1, 1)
            idx_wait(0, 0)
            gat_start(0, 0, choff)
            idx_start(2, 2)
            idx_wait(1, 1)
            gat_start(1, 1, choff)
            gat_wait(0)
            sc_start(0, 0)
            idx_start(3, 3)

            def full_iter(g, k4, k2):
                idx_wait(g, k4)
                sc_wait((k4 + 2) % 4, k2)
                gat_start(k4, k2, choff)
                gat_wait(1 - k2)
                sc_start((k4 + 3) % 4, 1 - k2)

            def lbody(i, _):
                for kk in range(4):
                    g = 2 + i * 4 + kk
                    full_iter(g, (2 + kk) % 4, kk % 2)
                    idx_start(g + 2, kk)
                return 0
            lax.fori_loop(0, 96, lbody, 0)

            for g in (386, 387, 388, 389, 390):
                full_iter(g, g % 4, g % 2)
                if g + 2 <= 390:
                    idx_start(g + 2, (g + 2) % 4)
            gat_wait(0)
            sc_start(2, 0)
            sc_wait(1, 1)
            sc_wait(2, 0)

            plsc.subcore_barrier()
            pltpu.sync_copy(acc.at[pl.ds(s * RPT, RPT), :],
                            out_hbm.at[pl.ds(choff + s * RPT, RPT), :])
    return k


_spmm6 = _make_spmm(6)
_spmm8 = _make_spmm(8)
_spmm4 = _make_spmm(4)


@functools.partial(
    pl.kernel,
    out_type=jax.ShapeDtypeStruct((NG * FREL,), jnp.float32),
    mesh=plsc.VectorSubcoreMesh(core_axis_name="c", subcore_axis_name="s"),
    scratch_types=[
        pltpu.VMEM((144,), jnp.int32),
        pltpu.VMEM((5, 56, 128), jnp.float32),
        pltpu.VMEM((FREL,), jnp.float32),
        pltpu.SemaphoreType.DMA,
    ],
)
def _sc_segmax(h0_hbm, h1_hbm, h2_hbm, x1_hbm, x2_hbm, starts_hbm, out_hbm,
               startsv, rowbuf, accv, sem):
    c = lax.axis_index("c")
    s = lax.axis_index("s")
    w = s * 2 + c
    pltpu.sync_copy(starts_hbm, startsv)
    ninf = jnp.full((16,), -jnp.inf, jnp.float32)
    srcs = (h0_hbm, h1_hbm, h2_hbm, x1_hbm, x2_hbm)

    for k in range(4):
        g = w * 4 + k
        r0 = startsv[pl.ds(g, 16)][0]
        r1 = startsv[pl.ds(g + 1, 16)][0]

        def zf(i, _):
            accv[pl.ds(i * 16, 16)] = ninf
            return 0
        lax.fori_loop(0, FREL // 16, zf, 0)

        nblk = (r1 - r0 + 47) // 48

        def blk(b, _):
            rs = r0 + b * 48
            rsa = pl.multiple_of((rs // 8) * 8, 8)
            off = rs - rsa
            cnt = jnp.minimum(48, r1 - rs)
            for a in range(5):
                pltpu.async_copy(srcs[a].at[pl.ds(rsa, 56), :],
                                 rowbuf.at[a], sem)
            for a in range(5):
                pltpu.make_async_copy(srcs[a].at[pl.ds(rsa, 56), :],
                                      rowbuf.at[a], sem).wait()

            def row(i, _):
                ii = off + i
                for cc in range(8):
                    h0 = rowbuf[0, ii, pl.ds(cc * 16, 16)]
                    h1 = rowbuf[1, ii, pl.ds(cc * 16, 16)]
                    h2 = rowbuf[2, ii, pl.ds(cc * 16, 16)]
                    x1 = rowbuf[3, ii, pl.ds(cc * 16, 16)]
                    x2 = rowbuf[4, ii, pl.ds(cc * 16, 16)]
                    parts = (h0, h1, h2, h0 * h1 * h2, h0 + h1 + h2,
                             x1, x2, x2 + x1, x2 * x1)
                    for q in range(9):
                        sl = pl.ds(q * 128 + cc * 16, 16)
                        accv[sl] = jnp.maximum(accv[sl], parts[q])
                return 0
            lax.fori_loop(0, cnt, row, 0)
            return 0
        lax.fori_loop(0, nblk, blk, 0)
        pltpu.sync_copy(accv,
                        out_hbm.at[pl.ds(pl.multiple_of(g * FREL, 8), FREL)])


def _rowspec(w):
    return pl.BlockSpec((BPR, w), lambda i: (i, 0))


def _fullspec(shape):
    nd = len(shape)
    return pl.BlockSpec(shape, lambda i: (0,) * nd)


def _tca(xp, deg_col):
    def body(x_ref, d_ref, ya_ref):
        xb = x_ref[...]
        dis = lax.rsqrt(d_ref[...])
        u = dis * xb
        ya_ref[...] = jnp.stack(
            [u[:, 0:32], u[:, 32:64], u[:, 64:96],
             xb[:, 0:32], xb[:, 32:64], xb[:, 64:96]], axis=0)

    return pl.pallas_call(
        body, grid=(NP // BPR,),
        in_specs=[_rowspec(96), _rowspec(1)],
        out_specs=pl.BlockSpec((6, BPR, 32), lambda i: (0, i, 0)),
        out_shape=jax.ShapeDtypeStruct((6, NP, 32), jnp.float32),
    )(xp, deg_col)


def _sums_update(i, r, sums_ref):
    rid = i * BPR + lax.broadcasted_iota(jnp.int32, (BPR, 1), 0)
    rm = jnp.where(rid < N, r, 0.0)
    part = jnp.concatenate(
        [jnp.sum(rm, 0, keepdims=True), jnp.sum(rm * rm, 0, keepdims=True),
         jnp.zeros((6, DIM), jnp.float32)], axis=0)

    @pl.when(i == 0)
    def _():
        sums_ref[...] = part

    @pl.when(i > 0)
    def _():
        sums_ref[...] = sums_ref[...] + part


def _tcb(za, xp, deg_col, w1p, b1, gw1p, gb1, gw2, gb2):
    def body(za_ref, x_ref, d_ref, w1_ref, b1_ref, gw1_ref, gb1_ref,
             gw2_ref, gb2_ref, xg1_ref, r0_ref, sums_ref):
        i = pl.program_id(0)
        za_ = za_ref[...]
        xb = x_ref[...]
        dis = lax.rsqrt(d_ref[...])
        v1 = jnp.concatenate([za_[0], za_[1], za_[2]], axis=1)
        t1 = v1 + dis * xb
        xg1 = jnp.maximum(
            dis * jnp.dot(t1, w1_ref[...],
                          preferred_element_type=jnp.float32) + b1_ref[...],
            0.0)
        g0 = jnp.concatenate([za_[3], za_[4], za_[5]], axis=1) + xb
        t = jnp.maximum(
            jnp.dot(g0, gw1_ref[...], preferred_element_type=jnp.float32)
            + gb1_ref[...], 0.0)
        r0 = jnp.maximum(
            jnp.dot(t, gw2_ref[...], preferred_element_type=jnp.float32)
            + gb2_ref[...], 0.0)
        xg1_ref[...] = xg1
        r0_ref[...] = r0
        _sums_update(i, r0, sums_ref)

    return pl.pallas_call(
        body, grid=(NP // BPR,),
        in_specs=[pl.BlockSpec((6, BPR, 32), lambda i: (0, i, 0)),
                  _rowspec(96), _rowspec(1),
                  _fullspec((96, DIM)), _fullspec((1, DIM)),
                  _fullspec((96, DIM)), _fullspec((1, DIM)),
                  _fullspec((DIM, DIM)), _fullspec((1, DIM))],
        out_specs=[_rowspec(DIM), _rowspec(DIM),
                   pl.BlockSpec((8, DIM), lambda i: (0, 0))],
        out_shape=[jax.ShapeDtypeStruct((NP, DIM), jnp.float32),
                   jax.ShapeDtypeStruct((NP, DIM), jnp.float32),
                   jax.ShapeDtypeStruct((8, DIM), jnp.float32)],
    )(za, xp, deg_col, w1p, b1, gw1p, gb1, gw2, gb2)


def _bn_coefs(sums_ref, g_ref, b_ref):
    sm = sums_ref[...]
    mean = sm[0:1] / N
    var = sm[1:2] / N - mean * mean
    sc = g_ref[...] * lax.rsqrt(var + 1e-5)
    sh = b_ref[...] - mean * sc
    return sc, sh


def _tcc(r0, sums0, xg1, deg_col, bng, bnb):
    def body(r0_ref, sums_ref, xg1_ref, d_ref, g_ref, b_ref,
             yb_ref, h0_ref):
        sc, sh = _bn_coefs(sums_ref, g_ref, b_ref)
        h0 = r0_ref[...] * sc + sh
        dis = lax.rsqrt(d_ref[...])
        u2 = dis * xg1_ref[...]
        yb_ref[...] = jnp.stack(
            [u2[:, 0:32], u2[:, 32:64], u2[:, 64:96], u2[:, 96:128],
             h0[:, 0:32], h0[:, 32:64], h0[:, 64:96], h0[:, 96:128]], axis=0)
        h0_ref[...] = h0

    return pl.pallas_call(
        body, grid=(NP // BPR,),
        in_specs=[_rowspec(DIM), _fullspec((8, DIM)), _rowspec(DIM),
                  _rowspec(1), _fullspec((1, DIM)), _fullspec((1, DIM))],
        out_specs=[pl.BlockSpec((8, BPR, 32), lambda i: (0, i, 0)),
                   _rowspec(DIM)],
        out_shape=[jax.ShapeDtypeStruct((8, NP, 32), jnp.float32),
                   jax.ShapeDtypeStruct((NP, DIM), jnp.float32)],
    )(r0, sums0, xg1, deg_col, bng, bnb)


def _tcd(zb, xg1, h0, deg_col, w2, b2, gw1, gb1, gw2, gb2):
    def body(zb_ref, xg1_ref, h0_ref, d_ref, w2_ref, b2_ref, gw1_ref,
             gb1_ref, gw2_ref, gb2_ref, xg2_ref, r1_ref, sums_ref):
        i = pl.program_id(0)
        zb_ = zb_ref[...]
        au2 = jnp.concatenate([zb_[0], zb_[1], zb_[2], zb_[3]], axis=1)
        dis = lax.rsqrt(d_ref[...])
        xg1_ = xg1_ref[...]
        t2 = au2 + dis * xg1_
        xg2 = jnp.maximum(
            dis * jnp.dot(t2, w2_ref[...],
                          preferred_element_type=jnp.float32) + b2_ref[...],
            0.0)
        ah0 = jnp.concatenate([zb_[4], zb_[5], zb_[6], zb_[7]], axis=1)
        g1 = ah0 + h0_ref[...]
        t = jnp.maximum(
            jnp.dot(g1, gw1_ref[...], preferred_element_type=jnp.float32)
            + gb1_ref[...], 0.0)
        r1 = jnp.maximum(
            jnp.dot(t, gw2_ref[...], preferred_element_type=jnp.float32)
            + gb2_ref[...], 0.0)
        xg2_ref[...] = xg2
        r1_ref[...] = r1
        _sums_update(i, r1, sums_ref)

    return pl.pallas_call(
        body, grid=(NP // BPR,),
        in_specs=[pl.BlockSpec((8, BPR, 32), lambda i: (0, i, 0)),
                  _rowspec(DIM), _rowspec(DIM), _rowspec(1),
                  _fullspec((DIM, DIM)), _fullspec((1, DIM)),
                  _fullspec((DIM, DIM)), _fullspec((1, DIM)),
                  _fullspec((DIM, DIM)), _fullspec((1, DIM))],
        out_specs=[_rowspec(DIM), _rowspec(DIM),
                   pl.BlockSpec((8, DIM), lambda i: (0, 0))],
        out_shape=[jax.ShapeDtypeStruct((NP, DIM), jnp.float32),
                   jax.ShapeDtypeStruct((NP, DIM), jnp.float32),
                   jax.ShapeDtypeStruct((8, DIM), jnp.float32)],
    )(zb, xg1, h0, deg_col, w2, b2, gw1, gb1, gw2, gb2)


def _tce(r1, sums1, bng, bnb):
    def body(r1_ref, sums_ref, g_ref, b_ref, yc_ref, h1_ref):
        sc, sh = _bn_coefs(sums_ref, g_ref, b_ref)
        h1 = r1_ref[...] * sc + sh
        yc_ref[...] = jnp.stack(
            [h1[:, 0:32], h1[:, 32:64], h1[:, 64:96], h1[:, 96:128]], axis=0)
        h1_ref[...] = h1

    return pl.pallas_call(
        body, grid=(NP // BPR,),
        in_specs=[_rowspec(DIM), _fullspec((8, DIM)),
                  _fullspec((1, DIM)), _fullspec((1, DIM))],
        out_specs=[pl.BlockSpec((4, BPR, 32), lambda i: (0, i, 0)),
                   _rowspec(DIM)],
        out_shape=[jax.ShapeDtypeStruct((4, NP, 32), jnp.float32),
                   jax.ShapeDtypeStruct((NP, DIM), jnp.float32)],
    )(r1, sums1, bng, bnb)


def _tcf(zc, h1, gw1, gb1, gw2, gb2):
    def body(zc_ref, h1_ref, gw1_ref, gb1_ref, gw2_ref, gb2_ref,
             r2_ref, sums_ref):
        i = pl.program_id(0)
        zc_ = zc_ref[...]
        g2 = jnp.concatenate([zc_[0], zc_[1], zc_[2], zc_[3]], axis=1) \
            + h1_ref[...]
        t = jnp.maximum(
            jnp.dot(g2, gw1_ref[...], preferred_element_type=jnp.float32)
            + gb1_ref[...], 0.0)
        r2 = jnp.maximum(
            jnp.dot(t, gw2_ref[...], preferred_element_type=jnp.float32)
            + gb2_ref[...], 0.0)
        r2_ref[...] = r2
        _sums_update(i, r2, sums_ref)

    return pl.pallas_call(
        body, grid=(NP // BPR,),
        in_specs=[pl.BlockSpec((4, BPR, 32), lambda i: (0, i, 0)),
                  _rowspec(DIM),
                  _fullspec((DIM, DIM)), _fullspec((1, DIM)),
                  _fullspec((DIM, DIM)), _fullspec((1, DIM))],
        out_specs=[_rowspec(DIM), pl.BlockSpec((8, DIM), lambda i: (0, 0))],
        out_shape=[jax.ShapeDtypeStruct((NP, DIM), jnp.float32),
                   jax.ShapeDtypeStruct((8, DIM), jnp.float32)],
    )(zc, h1, gw1, gb1, gw2, gb2)


def _tcg(r2, sums2, bng, bnb):
    def body(r2_ref, sums_ref, g_ref, b_ref, h2_ref):
        sc, sh = _bn_coefs(sums_ref, g_ref, b_ref)
        h2_ref[...] = r2_ref[...] * sc + sh

    return pl.pallas_call(
        body, grid=(NP // BPR,),
        in_specs=[_rowspec(DIM), _fullspec((8, DIM)),
                  _fullspec((1, DIM)), _fullspec((1, DIM))],
        out_specs=_rowspec(DIM),
        out_shape=jax.ShapeDtypeStruct((NP, DIM), jnp.float32),
    )(r2, sums2, bng, bnb)


def kernel(x, edge_index, batch, W_gcn1, b_gcn1, W_gcn2, b_gcn2,
           g0_W1, g0_b1, g0_W2, g0_b2, bn0_g, bn0_b,
           g1_W1, g1_b1, g1_W2, g1_b2, bn1_g, bn1_b,
           g2_W1, g2_b1, g2_W2, g2_b2, bn2_g, bn2_b):
    row = lambda v: v.reshape(1, DIM)
    padw = lambda w: jnp.pad(w, ((0, 96 - w.shape[0]), (0, 0)))

    xp = jnp.pad(x, ((0, NP - N), (0, 96 - 78)))
    batch_pad = jnp.pad(batch, (0, NP - N), constant_values=NG - 1)
    ei_pad = jnp.pad(edge_index, ((0, 0), (0, E_PAD - E)),
                     constant_values=NP - 1)

    deg, starts = _sc_prep(ei_pad, batch_pad)
    deg_col = deg.reshape(NP, 1)

    ya = _tca(xp, deg_col)
    za = _spmm6(ya.reshape(6 * NP, 32), ei_pad).reshape(6, NP, 32)
    xg1, r0, sums0 = _tcb(za, xp, deg_col, padw(W_gcn1), row(b_gcn1),
                          padw(g0_W1), row(g0_b1), g0_W2, row(g0_b2))
    yb, h0 = _tcc(r0, sums0, xg1, deg_col, row(bn0_g), row(bn0_b))
    zb = _spmm8(yb.reshape(8 * NP, 32), ei_pad).reshape(8, NP, 32)
    xg2, r1, sums1 = _tcd(zb, xg1, h0, deg_col, W_gcn2, row(b_gcn2),
                          g1_W1, row(g1_b1), g1_W2, row(g1_b2))
    yc, h1 = _tce(r1, sums1, row(bn1_g), row(bn1_b))
    zc = _spmm4(yc.reshape(4 * NP, 32), ei_pad).reshape(4, NP, 32)
    r2, sums2 = _tcf(zc, h1, g2_W1, row(g2_b1), g2_W2, row(g2_b2))
    h2 = _tcg(r2, sums2, row(bn2_g), row(bn2_b))
    out = _sc_segmax(h0, h1, h2, xg1, xg2, starts)
    return out.reshape(NG, FREL)


# speedup vs baseline: 8.0985x; 1.0087x over previous
import functools

import jax
import jax.numpy as jnp
from jax import lax
from jax.experimental import pallas as pl
from jax.experimental.pallas import tpu as pltpu
from jax.experimental.pallas import tpu_sc as plsc

N = 50000
E = 800000
DIM = 128
NG = 128
NP = 50176
RPT = NP // 16
NBLK = E // 128
BPR = 392
FREL = 9 * DIM


@functools.partial(
    pl.kernel,
    out_type=(
        jax.ShapeDtypeStruct((NP,), jnp.float32),
        jax.ShapeDtypeStruct((144,), jnp.int32),
    ),
    mesh=plsc.VectorSubcoreMesh(core_axis_name="c", subcore_axis_name="s"),
    compiler_params=pltpu.CompilerParams(needs_layout_passes=False),
    scratch_types=[
        pltpu.VMEM((NP,), jnp.float32),
        pltpu.VMEM((2048,), jnp.float32),
        pltpu.VMEM((128,), jnp.float32),
        pltpu.VMEM((2, 2048), jnp.int32),
        pltpu.VMEM((RPT,), jnp.int32),
        pltpu.VMEM((128,), jnp.int32),
        pltpu.VMEM((2048,), jnp.int32),
        pltpu.VMEM((144,), jnp.int32),
        pltpu.VMEM_SHARED((16 * NP,), jnp.float32),
        pltpu.VMEM_SHARED((2048,), jnp.int32),
        pltpu.SemaphoreType.DMA,
    ],
)
def _sc_prep(ei_hbm, batch_hbm, deg_out, starts_out,
             histv, stage, outb, idxv, bstage, bhist, bstage16, startsv,
             sh_deg, sh_b, csem):
    c = lax.axis_index("c")
    s = lax.axis_index("s")
    zero16 = jnp.zeros((16,), jnp.float32)
    one16 = jnp.ones((16,), jnp.float32)
    izero16 = jnp.zeros((16,), jnp.int32)
    ione16 = jnp.ones((16,), jnp.int32)

    @pl.when(c == 0)
    def _():
        def zb(i, _):
            histv[pl.ds(i * 16, 16)] = zero16
            return 0
        lax.fori_loop(0, NP // 16, zb, 0)
        nbt2 = jnp.where(s < 7, 25, 24)

        def eb(b, _):
            bid = b * 16 + s
            nv = jnp.where(bid == 390, 80, 128)

            boff = pl.multiple_of(bid * 2048, 128)

            @pl.when(bid == 390)
            def _():
                pltpu.sync_copy(ei_hbm.at[:, pl.ds(boff, 1280)],
                                idxv.at[:, pl.ds(0, 1280)])

            @pl.when(bid < 390)
            def _():
                pltpu.sync_copy(ei_hbm.at[:, pl.ds(boff, 2048)], idxv)

            def inner(j, _):
                v = idxv[1, pl.ds(j * 16, 16)]
                plsc.addupdate_scatter(histv, [v], one16)
                return 0
            lax.fori_loop(0, nv, inner, 0)
            return 0
        lax.fori_loop(0, nbt2, eb, 0)
        pltpu.sync_copy(histv, sh_deg.at[pl.ds(s * NP, NP)])
        plsc.subcore_barrier()
        ncb = jnp.where(s < 8, 25, 24)

        def comb(b, _):
            bid = b * 16 + s
            for r in range(16):
                so = pl.multiple_of(r * NP + bid * 128, 8)
                pltpu.async_copy(sh_deg.at[pl.ds(so, 128)],
                                 stage.at[pl.ds(r * 128, 128)], csem)
            for r in range(16):
                so = pl.multiple_of(r * NP + bid * 128, 8)
                pltpu.make_async_copy(
                    sh_deg.at[pl.ds(so, 128)],
                    stage.at[pl.ds(r * 128, 128)], csem).wait()
            for j in range(8):
                t = jnp.full((16,), 1.0, jnp.float32)
                for r in range(16):
                    t = t + stage[pl.ds(r * 128 + j * 16, 16)]
                outb[pl.ds(j * 16, 16)] = t
            pltpu.sync_copy(
                outb, deg_out.at[pl.ds(pl.multiple_of(bid * 128, 8), 128)])
            return 0
        lax.fori_loop(0, ncb, comb, 0)

    @pl.when(c == 1)
    def _():
        for j in range(8):
            bhist[pl.ds(j * 16, 16)] = izero16
        pltpu.sync_copy(batch_hbm.at[pl.ds(s * RPT, RPT)], bstage)
        def inner(j, _):
            v = bstage[pl.ds(j * 16, 16)]
            plsc.addupdate_scatter(bhist, [v], ione16)
            return 0
        lax.fori_loop(0, RPT // 16, inner, 0)
        pltpu.sync_copy(bhist, sh_b.at[pl.ds(s * 128, 128)])
        plsc.subcore_barrier()

        @pl.when(s == 0)
        def _():
            pltpu.sync_copy(sh_b, bstage16)
            carry = jnp.int32(0)
            for j in range(8):
                t = bstage16[pl.ds(j * 16, 16)]
                for r in range(1, 16):
                    t = t + bstage16[pl.ds(r * 128 + j * 16, 16)]
                incl = plsc.cumsum(t)
                startsv[pl.ds(j * 16, 16)] = incl - t + carry
                carry = carry + jnp.sum(t, axis=0)
            startsv[pl.ds(128, 16)] = jnp.full((16,), N, jnp.int32)
            pltpu.sync_copy(startsv, starts_out)


NBE = 392
NSB = NBE // 8
E_PAD = 16 * NBE * 128


def _make_spmm(n_chunks):
    cpc = n_chunks // 2

    @functools.partial(
        pl.kernel,
        out_type=jax.ShapeDtypeStruct((n_chunks * NP, 32), jnp.float32),
        mesh=plsc.VectorSubcoreMesh(core_axis_name="c", subcore_axis_name="s"),
        compiler_params=pltpu.CompilerParams(use_tc_tiling_on_sc=False),
        scratch_types=[
            pltpu.VMEM_SHARED((NP, 32), jnp.float32),
            pltpu.VMEM((2, 8, 128), jnp.int32),
            pltpu.VMEM((2, 1024), jnp.int32),
            pltpu.VMEM((4, 128), jnp.int32),
            pltpu.VMEM((4, 128, 32), jnp.float32),
            pltpu.VMEM((196, 32), jnp.float32),
            pltpu.SemaphoreType.DMA, pltpu.SemaphoreType.DMA,
            pltpu.SemaphoreType.DMA, pltpu.SemaphoreType.DMA,
            pltpu.SemaphoreType.DMA, pltpu.SemaphoreType.DMA,
            pltpu.SemaphoreType.DMA, pltpu.SemaphoreType.DMA,
            pltpu.SemaphoreType.DMA, pltpu.SemaphoreType.DMA,
        ],
    )
    def k(y_hbm, src_hbm, dst_hbm, out_hbm, acc, dbuf, sbuf, gbuf, rows, zv,
          is0, is1, gs0, gs1, gs2, gs3, ss0, ss1, ss2, ss3):
        c = lax.axis_index("c")
        s = lax.axis_index("s")
        isems = [is0, is1]
        gsems = [gs0, gs1, gs2, gs3]
        ssems = [ss0, ss1, ss2, ss3]
        zero16 = jnp.zeros((16,), jnp.float32)

        def zfill(i, _):
            zv[i // 2, pl.ds((i % 2) * 16, 16)] = zero16
            return 0
        lax.fori_loop(0, 196 * 2, zfill, 0)

        def islices(sb):
            ro = pl.multiple_of(s * NBE + sb * 8, 8)
            co = pl.multiple_of((s * NBE + sb * 8) * 128, 128)
            return dst_hbm.at[pl.ds(ro, 8), :], src_hbm.at[pl.ds(co, 1024)]

        def idx_start(sb, p):
            dsl, ssl = islices(sb)
            pltpu.async_copy(dsl, dbuf.at[p], isems[p])
            pltpu.async_copy(ssl, sbuf.at[p], isems[p])

        def idx_wait(sb, p):
            dsl, ssl = islices(sb)
            pltpu.make_async_copy(dsl, dbuf.at[p], isems[p]).wait()
            pltpu.make_async_copy(ssl, sbuf.at[p], isems[p]).wait()

        def gat_start(j, p, choff):
            k4 = j % 4
            for l in range(8):
                gbuf[k4, pl.ds(l * 16, 16)] = (
                    sbuf[p, pl.ds(j * 128 + l * 16, 16)] + choff)
            pltpu.async_copy(y_hbm.at[gbuf.at[k4]], rows.at[k4], gsems[k4])

        def gat_wait(k4):
            pltpu.make_async_copy(y_hbm.at[gbuf.at[k4]], rows.at[k4],
                                  gsems[k4]).wait()

        def sc_start(j, p):
            k4 = j % 4
            pltpu.async_copy(rows.at[k4], acc.at[dbuf.at[p, j]], ssems[k4],
                             add=True)

        def sc_wait(j, p):
            k4 = j % 4
            pltpu.make_async_copy(rows.at[k4], acc.at[dbuf.at[p, j]],
                                  ssems[k4]).wait()

        def sbody(sb, p, choff, first, last):
            if first:
                idx_wait(sb, p)
            for j in range(8):
                if first:
                    if j >= 4:
                        sc_wait(j - 4, p)
                else:
                    if j == 0:
                        idx_wait(sb, p)
                        sc_wait(4, 1 - p)
                    elif j < 4:
                        sc_wait(j + 4, 1 - p)
                    else:
                        sc_wait(j - 4, p)
                if j == 4 and not last:
                    idx_start(sb + 1, 1 - p)
                gat_start(j, p, choff)
                if first:
                    if j > 0:
                        gat_wait((j - 1) % 4)
                        sc_start(j - 1, p)
                else:
                    if j == 0:
                        gat_wait(3)
                        sc_start(7, 1 - p)
                    else:
                        gat_wait((j - 1) % 4)
                        sc_start(j - 1, p)

        for ci in range(cpc):
            choff = (c * cpc + ci) * NP
            for q in range(16):
                pltpu.sync_copy(zv, acc.at[pl.ds(s * RPT + q * 196, 196), :])
            plsc.subcore_barrier()

            idx_start(0, 0)
            sbody(0, 0, choff, True, False)

            def lbody(i, _):
                sb = 1 + i * 2
                sbody(sb, 1, choff, False, False)
                sbody(sb + 1, 0, choff, False, False)
                return 0
            lax.fori_loop(0, 23, lbody, 0)
            sbody(47, 1, choff, False, False)
            sbody(48, 0, choff, False, True)
            gat_wait(3)
            sc_start(7, 0)
            for j in (4, 5, 6, 7):
                sc_wait(j, 0)

            plsc.subcore_barrier()
            pltpu.sync_copy(acc.at[pl.ds(s * RPT, RPT), :],
                            out_hbm.at[pl.ds(choff + s * RPT, RPT), :])
    return k


_spmm6 = _make_spmm(6)
_spmm8 = _make_spmm(8)
_spmm4 = _make_spmm(4)


@functools.partial(
    pl.kernel,
    out_type=jax.ShapeDtypeStruct((NG * FREL,), jnp.float32),
    mesh=plsc.VectorSubcoreMesh(core_axis_name="c", subcore_axis_name="s"),
    scratch_types=[
        pltpu.VMEM((144,), jnp.int32),
        pltpu.VMEM((5, 56, 128), jnp.float32),
        pltpu.VMEM((FREL,), jnp.float32),
        pltpu.SemaphoreType.DMA,
    ],
)
def _sc_segmax(h0_hbm, h1_hbm, h2_hbm, x1_hbm, x2_hbm, starts_hbm, out_hbm,
               startsv, rowbuf, accv, sem):
    c = lax.axis_index("c")
    s = lax.axis_index("s")
    w = s * 2 + c
    pltpu.sync_copy(starts_hbm, startsv)
    ninf = jnp.full((16,), -jnp.inf, jnp.float32)
    srcs = (h0_hbm, h1_hbm, h2_hbm, x1_hbm, x2_hbm)

    for k in range(4):
        g = w * 4 + k
        r0 = startsv[pl.ds(g, 16)][0]
        r1 = startsv[pl.ds(g + 1, 16)][0]

        def zf(i, _):
            accv[pl.ds(i * 16, 16)] = ninf
            return 0
        lax.fori_loop(0, FREL // 16, zf, 0)

        nblk = (r1 - r0 + 47) // 48

        def blk(b, _):
            rs = r0 + b * 48
            rsa = pl.multiple_of((rs // 8) * 8, 8)
            off = rs - rsa
            cnt = jnp.minimum(48, r1 - rs)
            for a in range(5):
                pltpu.async_copy(srcs[a].at[pl.ds(rsa, 56), :],
                                 rowbuf.at[a], sem)
            for a in range(5):
                pltpu.make_async_copy(srcs[a].at[pl.ds(rsa, 56), :],
                                      rowbuf.at[a], sem).wait()

            def row(i, _):
                ii = off + i
                for cc in range(8):
                    h0 = rowbuf[0, ii, pl.ds(cc * 16, 16)]
                    h1 = rowbuf[1, ii, pl.ds(cc * 16, 16)]
                    h2 = rowbuf[2, ii, pl.ds(cc * 16, 16)]
                    x1 = rowbuf[3, ii, pl.ds(cc * 16, 16)]
                    x2 = rowbuf[4, ii, pl.ds(cc * 16, 16)]
                    parts = (h0, h1, h2, h0 * h1 * h2, h0 + h1 + h2,
                             x1, x2, x2 + x1, x2 * x1)
                    for q in range(9):
                        sl = pl.ds(q * 128 + cc * 16, 16)
                        accv[sl] = jnp.maximum(accv[sl], parts[q])
                return 0
            lax.fori_loop(0, cnt, row, 0)
            return 0
        lax.fori_loop(0, nblk, blk, 0)
        pltpu.sync_copy(accv,
                        out_hbm.at[pl.ds(pl.multiple_of(g * FREL, 8), FREL)])


def _rowspec(w):
    return pl.BlockSpec((BPR, w), lambda i: (i, 0))


def _fullspec(shape):
    nd = len(shape)
    return pl.BlockSpec(shape, lambda i: (0,) * nd)


def _tca(xp, deg_col):
    def body(x_ref, d_ref, ya_ref):
        xb = x_ref[...]
        dis = lax.rsqrt(d_ref[...])
        u = dis * xb
        ya_ref[...] = jnp.stack(
            [u[:, 0:32], u[:, 32:64], u[:, 64:96],
             xb[:, 0:32], xb[:, 32:64], xb[:, 64:96]], axis=0)

    return pl.pallas_call(
        body, grid=(NP // BPR,),
        in_specs=[_rowspec(96), _rowspec(1)],
        out_specs=pl.BlockSpec((6, BPR, 32), lambda i: (0, i, 0)),
        out_shape=jax.ShapeDtypeStruct((6, NP, 32), jnp.float32),
    )(xp, deg_col)


def _sums_update(i, r, sums_ref):
    rid = i * BPR + lax.broadcasted_iota(jnp.int32, (BPR, 1), 0)
    rm = jnp.where(rid < N, r, 0.0)
    part = jnp.concatenate(
        [jnp.sum(rm, 0, keepdims=True), jnp.sum(rm * rm, 0, keepdims=True),
         jnp.zeros((6, DIM), jnp.float32)], axis=0)

    @pl.when(i == 0)
    def _():
        sums_ref[...] = part

    @pl.when(i > 0)
    def _():
        sums_ref[...] = sums_ref[...] + part


def _tcb(za, xp, deg_col, w1p, b1, gw1p, gb1, gw2, gb2):
    def body(za_ref, x_ref, d_ref, w1_ref, b1_ref, gw1_ref, gb1_ref,
             gw2_ref, gb2_ref, xg1_ref, r0_ref, sums_ref):
        i = pl.program_id(0)
        za_ = za_ref[...]
        xb = x_ref[...]
        dis = lax.rsqrt(d_ref[...])
        v1 = jnp.concatenate([za_[0], za_[1], za_[2]], axis=1)
        t1 = v1 + dis * xb
        xg1 = jnp.maximum(
            dis * jnp.dot(t1, w1_ref[...],
                          preferred_element_type=jnp.float32) + b1_ref[...],
            0.0)
        g0 = jnp.concatenate([za_[3], za_[4], za_[5]], axis=1) + xb
        t = jnp.maximum(
            jnp.dot(g0, gw1_ref[...], preferred_element_type=jnp.float32)
            + gb1_ref[...], 0.0)
        r0 = jnp.maximum(
            jnp.dot(t, gw2_ref[...], preferred_element_type=jnp.float32)
            + gb2_ref[...], 0.0)
        xg1_ref[...] = xg1
        r0_ref[...] = r0
        _sums_update(i, r0, sums_ref)

    return pl.pallas_call(
        body, grid=(NP // BPR,),
        in_specs=[pl.BlockSpec((6, BPR, 32), lambda i: (0, i, 0)),
                  _rowspec(96), _rowspec(1),
                  _fullspec((96, DIM)), _fullspec((1, DIM)),
                  _fullspec((96, DIM)), _fullspec((1, DIM)),
                  _fullspec((DIM, DIM)), _fullspec((1, DIM))],
        out_specs=[_rowspec(DIM), _rowspec(DIM),
                   pl.BlockSpec((8, DIM), lambda i: (0, 0))],
        out_shape=[jax.ShapeDtypeStruct((NP, DIM), jnp.float32),
                   jax.ShapeDtypeStruct((NP, DIM), jnp.float32),
                   jax.ShapeDtypeStruct((8, DIM), jnp.float32)],
    )(za, xp, deg_col, w1p, b1, gw1p, gb1, gw2, gb2)


def _bn_coefs(sums_ref, g_ref, b_ref):
    sm = sums_ref[...]
    mean = sm[0:1] / N
    var = sm[1:2] / N - mean * mean
    sc = g_ref[...] * lax.rsqrt(var + 1e-5)
    sh = b_ref[...] - mean * sc
    return sc, sh


def _tcc(r0, sums0, xg1, deg_col, bng, bnb):
    def body(r0_ref, sums_ref, xg1_ref, d_ref, g_ref, b_ref,
             yb_ref, h0_ref):
        sc, sh = _bn_coefs(sums_ref, g_ref, b_ref)
        h0 = r0_ref[...] * sc + sh
        dis = lax.rsqrt(d_ref[...])
        u2 = dis * xg1_ref[...]
        yb_ref[...] = jnp.stack(
            [u2[:, 0:32], u2[:, 32:64], u2[:, 64:96], u2[:, 96:128],
             h0[:, 0:32], h0[:, 32:64], h0[:, 64:96], h0[:, 96:128]], axis=0)
        h0_ref[...] = h0

    return pl.pallas_call(
        body, grid=(NP // BPR,),
        in_specs=[_rowspec(DIM), _fullspec((8, DIM)), _rowspec(DIM),
                  _rowspec(1), _fullspec((1, DIM)), _fullspec((1, DIM))],
        out_specs=[pl.BlockSpec((8, BPR, 32), lambda i: (0, i, 0)),
                   _rowspec(DIM)],
        out_shape=[jax.ShapeDtypeStruct((8, NP, 32), jnp.float32),
                   jax.ShapeDtypeStruct((NP, DIM), jnp.float32)],
    )(r0, sums0, xg1, deg_col, bng, bnb)


def _tcd(zb, xg1, h0, deg_col, w2, b2, gw1, gb1, gw2, gb2):
    def body(zb_ref, xg1_ref, h0_ref, d_ref, w2_ref, b2_ref, gw1_ref,
             gb1_ref, gw2_ref, gb2_ref, xg2_ref, r1_ref, sums_ref):
        i = pl.program_id(0)
        zb_ = zb_ref[...]
        au2 = jnp.concatenate([zb_[0], zb_[1], zb_[2], zb_[3]], axis=1)
        dis = lax.rsqrt(d_ref[...])
        xg1_ = xg1_ref[...]
        t2 = au2 + dis * xg1_
        xg2 = jnp.maximum(
            dis * jnp.dot(t2, w2_ref[...],
                          preferred_element_type=jnp.float32) + b2_ref[...],
            0.0)
        ah0 = jnp.concatenate([zb_[4], zb_[5], zb_[6], zb_[7]], axis=1)
        g1 = ah0 + h0_ref[...]
        t = jnp.maximum(
            jnp.dot(g1, gw1_ref[...], preferred_element_type=jnp.float32)
            + gb1_ref[...], 0.0)
        r1 = jnp.maximum(
            jnp.dot(t, gw2_ref[...], preferred_element_type=jnp.float32)
            + gb2_ref[...], 0.0)
        xg2_ref[...] = xg2
        r1_ref[...] = r1
        _sums_update(i, r1, sums_ref)

    return pl.pallas_call(
        body, grid=(NP // BPR,),
        in_specs=[pl.BlockSpec((8, BPR, 32), lambda i: (0, i, 0)),
                  _rowspec(DIM), _rowspec(DIM), _rowspec(1),
                  _fullspec((DIM, DIM)), _fullspec((1, DIM)),
                  _fullspec((DIM, DIM)), _fullspec((1, DIM)),
                  _fullspec((DIM, DIM)), _fullspec((1, DIM))],
        out_specs=[_rowspec(DIM), _rowspec(DIM),
                   pl.BlockSpec((8, DIM), lambda i: (0, 0))],
        out_shape=[jax.ShapeDtypeStruct((NP, DIM), jnp.float32),
                   jax.ShapeDtypeStruct((NP, DIM), jnp.float32),
                   jax.ShapeDtypeStruct((8, DIM), jnp.float32)],
    )(zb, xg1, h0, deg_col, w2, b2, gw1, gb1, gw2, gb2)


def _tce(r1, sums1, bng, bnb):
    def body(r1_ref, sums_ref, g_ref, b_ref, yc_ref, h1_ref):
        sc, sh = _bn_coefs(sums_ref, g_ref, b_ref)
        h1 = r1_ref[...] * sc + sh
        yc_ref[...] = jnp.stack(
            [h1[:, 0:32], h1[:, 32:64], h1[:, 64:96], h1[:, 96:128]], axis=0)
        h1_ref[...] = h1

    return pl.pallas_call(
        body, grid=(NP // BPR,),
        in_specs=[_rowspec(DIM), _fullspec((8, DIM)),
                  _fullspec((1, DIM)), _fullspec((1, DIM))],
        out_specs=[pl.BlockSpec((4, BPR, 32), lambda i: (0, i, 0)),
                   _rowspec(DIM)],
        out_shape=[jax.ShapeDtypeStruct((4, NP, 32), jnp.float32),
                   jax.ShapeDtypeStruct((NP, DIM), jnp.float32)],
    )(r1, sums1, bng, bnb)


def _tcf(zc, h1, gw1, gb1, gw2, gb2):
    def body(zc_ref, h1_ref, gw1_ref, gb1_ref, gw2_ref, gb2_ref,
             r2_ref, sums_ref):
        i = pl.program_id(0)
        zc_ = zc_ref[...]
        g2 = jnp.concatenate([zc_[0], zc_[1], zc_[2], zc_[3]], axis=1) \
            + h1_ref[...]
        t = jnp.maximum(
            jnp.dot(g2, gw1_ref[...], preferred_element_type=jnp.float32)
            + gb1_ref[...], 0.0)
        r2 = jnp.maximum(
            jnp.dot(t, gw2_ref[...], preferred_element_type=jnp.float32)
            + gb2_ref[...], 0.0)
        r2_ref[...] = r2
        _sums_update(i, r2, sums_ref)

    return pl.pallas_call(
        body, grid=(NP // BPR,),
        in_specs=[pl.BlockSpec((4, BPR, 32), lambda i: (0, i, 0)),
                  _rowspec(DIM),
                  _fullspec((DIM, DIM)), _fullspec((1, DIM)),
                  _fullspec((DIM, DIM)), _fullspec((1, DIM))],
        out_specs=[_rowspec(DIM), pl.BlockSpec((8, DIM), lambda i: (0, 0))],
        out_shape=[jax.ShapeDtypeStruct((NP, DIM), jnp.float32),
                   jax.ShapeDtypeStruct((8, DIM), jnp.float32)],
    )(zc, h1, gw1, gb1, gw2, gb2)


def _tcg(r2, sums2, bng, bnb):
    def body(r2_ref, sums_ref, g_ref, b_ref, h2_ref):
        sc, sh = _bn_coefs(sums_ref, g_ref, b_ref)
        h2_ref[...] = r2_ref[...] * sc + sh

    return pl.pallas_call(
        body, grid=(NP // BPR,),
        in_specs=[_rowspec(DIM), _fullspec((8, DIM)),
                  _fullspec((1, DIM)), _fullspec((1, DIM))],
        out_specs=_rowspec(DIM),
        out_shape=jax.ShapeDtypeStruct((NP, DIM), jnp.float32),
    )(r2, sums2, bng, bnb)


def kernel(x, edge_index, batch, W_gcn1, b_gcn1, W_gcn2, b_gcn2,
           g0_W1, g0_b1, g0_W2, g0_b2, bn0_g, bn0_b,
           g1_W1, g1_b1, g1_W2, g1_b2, bn1_g, bn1_b,
           g2_W1, g2_b1, g2_W2, g2_b2, bn2_g, bn2_b):
    row = lambda v: v.reshape(1, DIM)
    padw = lambda w: jnp.pad(w, ((0, 96 - w.shape[0]), (0, 0)))

    xp = jnp.pad(x, ((0, NP - N), (0, 96 - 78)))
    batch_pad = jnp.pad(batch, (0, NP - N), constant_values=NG - 1)
    ei_pad = jnp.pad(edge_index, ((0, 0), (0, E_PAD - E)),
                     constant_values=NP - 1)
    src_flat = ei_pad[0]
    dst2d = ei_pad[1].reshape(16 * NBE, 128)

    deg, starts = _sc_prep(ei_pad, batch_pad)
    deg_col = deg.reshape(NP, 1)

    ya = _tca(xp, deg_col)
    za = _spmm6(ya.reshape(6 * NP, 32), src_flat, dst2d).reshape(6, NP, 32)
    xg1, r0, sums0 = _tcb(za, xp, deg_col, padw(W_gcn1), row(b_gcn1),
                          padw(g0_W1), row(g0_b1), g0_W2, row(g0_b2))
    yb, h0 = _tcc(r0, sums0, xg1, deg_col, row(bn0_g), row(bn0_b))
    zb = _spmm8(yb.reshape(8 * NP, 32), src_flat, dst2d).reshape(8, NP, 32)
    xg2, r1, sums1 = _tcd(zb, xg1, h0, deg_col, W_gcn2, row(b_gcn2),
                          g1_W1, row(g1_b1), g1_W2, row(g1_b2))
    yc, h1 = _tce(r1, sums1, row(bn1_g), row(bn1_b))
    zc = _spmm4(yc.reshape(4 * NP, 32), src_flat, dst2d).reshape(4, NP, 32)
    r2, sums2 = _tcf(zc, h1, g2_W1, row(g2_b1), g2_W2, row(g2_b2))
    h2 = _tcg(r2, sums2, row(bn2_g), row(bn2_b))
    out = _sc_segmax(h0, h1, h2, xg1, xg2, starts)
    return out.reshape(NG, FREL)


# speedup vs baseline: 8.1259x; 1.0034x over previous
import functools

import jax
import jax.numpy as jnp
from jax import lax
from jax.experimental import pallas as pl
from jax.experimental.pallas import tpu as pltpu
from jax.experimental.pallas import tpu_sc as plsc

N = 50000
E = 800000
DIM = 128
NG = 128
NP = 50176
RPT = NP // 16
NBLK = E // 128
BPR = 392
FREL = 9 * DIM


@functools.partial(
    pl.kernel,
    out_type=(
        jax.ShapeDtypeStruct((NP,), jnp.float32),
        jax.ShapeDtypeStruct((144,), jnp.int32),
    ),
    mesh=plsc.VectorSubcoreMesh(core_axis_name="c", subcore_axis_name="s"),
    compiler_params=pltpu.CompilerParams(needs_layout_passes=False),
    scratch_types=[
        pltpu.VMEM((NP,), jnp.float32),
        pltpu.VMEM((2048,), jnp.float32),
        pltpu.VMEM((128,), jnp.float32),
        pltpu.VMEM((2, 2048), jnp.int32),
        pltpu.VMEM((RPT,), jnp.int32),
        pltpu.VMEM((128,), jnp.int32),
        pltpu.VMEM((2048,), jnp.int32),
        pltpu.VMEM((144,), jnp.int32),
        pltpu.VMEM_SHARED((16 * NP,), jnp.float32),
        pltpu.VMEM_SHARED((2048,), jnp.int32),
        pltpu.SemaphoreType.DMA,
    ],
)
def _sc_prep(ei_hbm, batch_hbm, deg_out, starts_out,
             histv, stage, outb, idxv, bstage, bhist, bstage16, startsv,
             sh_deg, sh_b, csem):
    c = lax.axis_index("c")
    s = lax.axis_index("s")
    zero16 = jnp.zeros((16,), jnp.float32)
    one16 = jnp.ones((16,), jnp.float32)
    izero16 = jnp.zeros((16,), jnp.int32)
    ione16 = jnp.ones((16,), jnp.int32)

    @pl.when(c == 0)
    def _():
        def zb(i, _):
            histv[pl.ds(i * 16, 16)] = zero16
            return 0
        lax.fori_loop(0, NP // 16, zb, 0)
        nbt2 = jnp.where(s < 7, 25, 24)

        def eb(b, _):
            bid = b * 16 + s
            nv = jnp.where(bid == 390, 80, 128)

            boff = pl.multiple_of(bid * 2048, 128)

            @pl.when(bid == 390)
            def _():
                pltpu.sync_copy(ei_hbm.at[:, pl.ds(boff, 1280)],
                                idxv.at[:, pl.ds(0, 1280)])

            @pl.when(bid < 390)
            def _():
                pltpu.sync_copy(ei_hbm.at[:, pl.ds(boff, 2048)], idxv)

            def inner(j, _):
                v = idxv[1, pl.ds(j * 16, 16)]
                plsc.addupdate_scatter(histv, [v], one16)
                return 0
            lax.fori_loop(0, nv, inner, 0)
            return 0
        lax.fori_loop(0, nbt2, eb, 0)
        pltpu.sync_copy(histv, sh_deg.at[pl.ds(s * NP, NP)])
        plsc.subcore_barrier()
        ncb = jnp.where(s < 8, 25, 24)

        def comb(b, _):
            bid = b * 16 + s
            for r in range(16):
                so = pl.multiple_of(r * NP + bid * 128, 8)
                pltpu.async_copy(sh_deg.at[pl.ds(so, 128)],
                                 stage.at[pl.ds(r * 128, 128)], csem)
            for r in range(16):
                so = pl.multiple_of(r * NP + bid * 128, 8)
                pltpu.make_async_copy(
                    sh_deg.at[pl.ds(so, 128)],
                    stage.at[pl.ds(r * 128, 128)], csem).wait()
            for j in range(8):
                t = jnp.full((16,), 1.0, jnp.float32)
                for r in range(16):
                    t = t + stage[pl.ds(r * 128 + j * 16, 16)]
                outb[pl.ds(j * 16, 16)] = t
            pltpu.sync_copy(
                outb, deg_out.at[pl.ds(pl.multiple_of(bid * 128, 8), 128)])
            return 0
        lax.fori_loop(0, ncb, comb, 0)

    @pl.when(c == 1)
    def _():
        for j in range(8):
            bhist[pl.ds(j * 16, 16)] = izero16
        pltpu.sync_copy(batch_hbm.at[pl.ds(s * RPT, RPT)], bstage)
        def inner(j, _):
            v = bstage[pl.ds(j * 16, 16)]
            plsc.addupdate_scatter(bhist, [v], ione16)
            return 0
        lax.fori_loop(0, RPT // 16, inner, 0)
        pltpu.sync_copy(bhist, sh_b.at[pl.ds(s * 128, 128)])
        plsc.subcore_barrier()

        @pl.when(s == 0)
        def _():
            pltpu.sync_copy(sh_b, bstage16)
            carry = jnp.int32(0)
            for j in range(8):
                t = bstage16[pl.ds(j * 16, 16)]
                for r in range(1, 16):
                    t = t + bstage16[pl.ds(r * 128 + j * 16, 16)]
                incl = plsc.cumsum(t)
                startsv[pl.ds(j * 16, 16)] = incl - t + carry
                carry = carry + jnp.sum(t, axis=0)
            startsv[pl.ds(128, 16)] = jnp.full((16,), N, jnp.int32)
            pltpu.sync_copy(startsv, starts_out)


NBE = 392
NSB = NBE // 8
E_PAD = 16 * NBE * 128


def _make_spmm(n_chunks):
    cpc = n_chunks // 2

    @functools.partial(
        pl.kernel,
        out_type=jax.ShapeDtypeStruct((n_chunks * NP, 32), jnp.float32),
        mesh=plsc.VectorSubcoreMesh(core_axis_name="c", subcore_axis_name="s"),
        compiler_params=pltpu.CompilerParams(use_tc_tiling_on_sc=False),
        scratch_types=[
            pltpu.VMEM_SHARED((NP, 32), jnp.float32),
            pltpu.VMEM((2, 8, 128), jnp.int32),
            pltpu.VMEM((2, 1024), jnp.int32),
            pltpu.VMEM((4, 128), jnp.int32),
            pltpu.VMEM((4, 128, 32), jnp.float32),
            pltpu.VMEM((196, 32), jnp.float32),
            pltpu.SemaphoreType.DMA, pltpu.SemaphoreType.DMA,
            pltpu.SemaphoreType.DMA, pltpu.SemaphoreType.DMA,
            pltpu.SemaphoreType.DMA, pltpu.SemaphoreType.DMA,
            pltpu.SemaphoreType.DMA, pltpu.SemaphoreType.DMA,
            pltpu.SemaphoreType.DMA, pltpu.SemaphoreType.DMA,
        ],
    )
    def k(y_hbm, src_hbm, dst_hbm, out_hbm, acc, dbuf, sbuf, gbuf, rows, zv,
          is0, is1, gs0, gs1, gs2, gs3, ss0, ss1, ss2, ss3):
        c = lax.axis_index("c")
        s = lax.axis_index("s")
        isems = [is0, is1]
        gsems = [gs0, gs1, gs2, gs3]
        ssems = [ss0, ss1, ss2, ss3]
        zero16 = jnp.zeros((16,), jnp.float32)

        def zfill(i, _):
            zv[i // 2, pl.ds((i % 2) * 16, 16)] = zero16
            return 0
        lax.fori_loop(0, 196 * 2, zfill, 0)

        def islices(sb):
            ro = pl.multiple_of(s * NBE + sb * 8, 8)
            co = pl.multiple_of((s * NBE + sb * 8) * 128, 128)
            return dst_hbm.at[pl.ds(ro, 8), :], src_hbm.at[pl.ds(co, 1024)]

        def idx_start(sb, p):
            dsl, ssl = islices(sb)
            pltpu.async_copy(dsl, dbuf.at[p], isems[p])
            pltpu.async_copy(ssl, sbuf.at[p], isems[p])

        def idx_wait(sb, p):
            dsl, ssl = islices(sb)
            pltpu.make_async_copy(dsl, dbuf.at[p], isems[p]).wait()
            pltpu.make_async_copy(ssl, sbuf.at[p], isems[p]).wait()

        def gat_start(j, p, choff):
            k4 = j % 4
            for l in range(8):
                gbuf[k4, pl.ds(l * 16, 16)] = (
                    sbuf[p, pl.ds(j * 128 + l * 16, 16)] + choff)
            pltpu.async_copy(y_hbm.at[gbuf.at[k4]], rows.at[k4], gsems[k4])

        def gat_wait(k4):
            pltpu.make_async_copy(y_hbm.at[gbuf.at[k4]], rows.at[k4],
                                  gsems[k4]).wait()

        def sc_start(j, p):
            k4 = j % 4
            pltpu.async_copy(rows.at[k4], acc.at[dbuf.at[p, j]], ssems[k4],
                             add=True)

        def sc_wait(j, p):
            k4 = j % 4
            pltpu.make_async_copy(rows.at[k4], acc.at[dbuf.at[p, j]],
                                  ssems[k4]).wait()

        def sbody(sb, p, choff, first, last):
            if first:
                idx_wait(sb, p)
            for j in range(8):
                if first:
                    if j >= 4:
                        sc_wait(j - 4, p)
                else:
                    if j == 0:
                        idx_wait(sb, p)
                        sc_wait(4, 1 - p)
                    elif j < 4:
                        sc_wait(j + 4, 1 - p)
                    else:
                        sc_wait(j - 4, p)
                if j == 4 and not last:
                    idx_start(sb + 1, 1 - p)
                gat_start(j, p, choff)
                if first:
                    if j > 0:
                        gat_wait((j - 1) % 4)
                        sc_start(j - 1, p)
                else:
                    if j == 0:
                        gat_wait(3)
                        sc_start(7, 1 - p)
                    else:
                        gat_wait((j - 1) % 4)
                        sc_start(j - 1, p)

        for ci in range(cpc):
            choff = (c * cpc + ci) * NP
            for q in range(16):
                pltpu.sync_copy(zv, acc.at[pl.ds(s * RPT + q * 196, 196), :])
            plsc.subcore_barrier()

            idx_start(0, 0)
            sbody(0, 0, choff, True, False)

            def lbody(i, _):
                sb = 1 + i * 2
                sbody(sb, 1, choff, False, False)
                sbody(sb + 1, 0, choff, False, False)
                return 0
            lax.fori_loop(0, 23, lbody, 0)
            sbody(47, 1, choff, False, False)
            sbody(48, 0, choff, False, True)
            gat_wait(3)
            sc_start(7, 0)
            for j in (4, 5, 6, 7):
                sc_wait(j, 0)

            plsc.subcore_barrier()
            pltpu.sync_copy(acc.at[pl.ds(s * RPT, RPT), :],
                            out_hbm.at[pl.ds(choff + s * RPT, RPT), :])
    return k


_spmm6 = _make_spmm(6)
_spmm8 = _make_spmm(8)
_spmm4 = _make_spmm(4)


@functools.partial(
    pl.kernel,
    out_type=jax.ShapeDtypeStruct((NG * FREL,), jnp.float32),
    mesh=plsc.VectorSubcoreMesh(core_axis_name="c", subcore_axis_name="s"),
    scratch_types=[
        pltpu.VMEM((144,), jnp.int32),
        pltpu.VMEM((5, 56, 128), jnp.float32),
        pltpu.VMEM((FREL,), jnp.float32),
        pltpu.SemaphoreType.DMA,
    ],
)
def _sc_segmax(h0_hbm, h1_hbm, h2_hbm, x1_hbm, x2_hbm, starts_hbm, out_hbm,
               startsv, rowbuf, accv, sem):
    c = lax.axis_index("c")
    s = lax.axis_index("s")
    w = s * 2 + c
    pltpu.sync_copy(starts_hbm, startsv)
    ninf = jnp.full((16,), -jnp.inf, jnp.float32)
    srcs = (h0_hbm, h1_hbm, h2_hbm, x1_hbm, x2_hbm)

    for k in range(4):
        g = w * 4 + k
        r0 = startsv[pl.ds(g, 16)][0]
        r1 = startsv[pl.ds(g + 1, 16)][0]

        def zf(i, _):
            accv[pl.ds(i * 16, 16)] = ninf
            return 0
        lax.fori_loop(0, FREL // 16, zf, 0)

        nblk = (r1 - r0 + 47) // 48

        def blk(b, _):
            rs = r0 + b * 48
            rsa = pl.multiple_of((rs // 8) * 8, 8)
            off = rs - rsa
            cnt = jnp.minimum(48, r1 - rs)
            for a in range(5):
                pltpu.async_copy(srcs[a].at[pl.ds(rsa, 56), :],
                                 rowbuf.at[a], sem)
            for a in range(5):
                pltpu.make_async_copy(srcs[a].at[pl.ds(rsa, 56), :],
                                      rowbuf.at[a], sem).wait()

            def row(i, _):
                ii = off + i
                for cc in range(8):
                    h0 = rowbuf[0, ii, pl.ds(cc * 16, 16)]
                    h1 = rowbuf[1, ii, pl.ds(cc * 16, 16)]
                    h2 = rowbuf[2, ii, pl.ds(cc * 16, 16)]
                    x1 = rowbuf[3, ii, pl.ds(cc * 16, 16)]
                    x2 = rowbuf[4, ii, pl.ds(cc * 16, 16)]
                    parts = (h0, h1, h2, h0 * h1 * h2, h0 + h1 + h2,
                             x1, x2, x2 + x1, x2 * x1)
                    for q in range(9):
                        sl = pl.ds(q * 128 + cc * 16, 16)
                        accv[sl] = jnp.maximum(accv[sl], parts[q])
                return 0
            lax.fori_loop(0, cnt, row, 0)
            return 0
        lax.fori_loop(0, nblk, blk, 0)
        pltpu.sync_copy(accv,
                        out_hbm.at[pl.ds(pl.multiple_of(g * FREL, 8), FREL)])


def _rowspec(w):
    return pl.BlockSpec((BPR, w), lambda i: (i, 0))


def _fullspec(shape):
    nd = len(shape)
    return pl.BlockSpec(shape, lambda i: (0,) * nd)


def _tca(xp, deg_col):
    def body(x_ref, d_ref, ya_ref):
        xb = x_ref[...]
        dis = lax.rsqrt(d_ref[...])
        u = dis * xb
        ya_ref[...] = jnp.stack(
            [u[:, 0:32], u[:, 32:64], u[:, 64:96],
             xb[:, 0:32], xb[:, 32:64], xb[:, 64:96]], axis=0)

    return pl.pallas_call(
        body, grid=(NP // BPR,),
        in_specs=[_rowspec(96), _rowspec(1)],
        out_specs=pl.BlockSpec((6, BPR, 32), lambda i: (0, i, 0)),
        out_shape=jax.ShapeDtypeStruct((6, NP, 32), jnp.float32),
    )(xp, deg_col)


def _sums_update(i, r, sums_ref):
    rid = i * BPR + lax.broadcasted_iota(jnp.int32, (BPR, 1), 0)
    rm = jnp.where(rid < N, r, 0.0)
    part = jnp.concatenate(
        [jnp.sum(rm, 0, keepdims=True), jnp.sum(rm * rm, 0, keepdims=True),
         jnp.zeros((6, DIM), jnp.float32)], axis=0)

    @pl.when(i == 0)
    def _():
        sums_ref[...] = part

    @pl.when(i > 0)
    def _():
        sums_ref[...] = sums_ref[...] + part


def _zspec(k):
    return pl.BlockSpec((BPR, 32), lambda i, _k=k: (_k * (NP // BPR) + i, 0))


def _tcb(za, xp, deg_col, w1p, b1, gw1p, gb1, gw2, gb2):
    def body(z0, z1, z2, z3, z4, z5, x_ref, d_ref, w1_ref, b1_ref,
             gw1_ref, gb1_ref, gw2_ref, gb2_ref, xg1_ref, r0_ref, sums_ref):
        i = pl.program_id(0)
        xb = x_ref[...]
        dis = lax.rsqrt(d_ref[...])
        v1 = jnp.concatenate([z0[...], z1[...], z2[...]], axis=1)
        t1 = v1 + dis * xb
        xg1 = jnp.maximum(
            dis * jnp.dot(t1, w1_ref[...],
                          preferred_element_type=jnp.float32) + b1_ref[...],
            0.0)
        g0 = jnp.concatenate([z3[...], z4[...], z5[...]], axis=1) + xb
        t = jnp.maximum(
            jnp.dot(g0, gw1_ref[...], preferred_element_type=jnp.float32)
            + gb1_ref[...], 0.0)
        r0 = jnp.maximum(
            jnp.dot(t, gw2_ref[...], preferred_element_type=jnp.float32)
            + gb2_ref[...], 0.0)
        xg1_ref[...] = xg1
        r0_ref[...] = r0
        _sums_update(i, r0, sums_ref)

    return pl.pallas_call(
        body, grid=(NP // BPR,),
        in_specs=[_zspec(0), _zspec(1), _zspec(2), _zspec(3), _zspec(4),
                  _zspec(5),
                  _rowspec(96), _rowspec(1),
                  _fullspec((96, DIM)), _fullspec((1, DIM)),
                  _fullspec((96, DIM)), _fullspec((1, DIM)),
                  _fullspec((DIM, DIM)), _fullspec((1, DIM))],
        out_specs=[_rowspec(DIM), _rowspec(DIM),
                   pl.BlockSpec((8, DIM), lambda i: (0, 0))],
        out_shape=[jax.ShapeDtypeStruct((NP, DIM), jnp.float32),
                   jax.ShapeDtypeStruct((NP, DIM), jnp.float32),
                   jax.ShapeDtypeStruct((8, DIM), jnp.float32)],
    )(za, za, za, za, za, za, xp, deg_col, w1p, b1, gw1p, gb1, gw2, gb2)


def _bn_coefs(sums_ref, g_ref, b_ref):
    sm = sums_ref[...]
    mean = sm[0:1] / N
    var = sm[1:2] / N - mean * mean
    sc = g_ref[...] * lax.rsqrt(var + 1e-5)
    sh = b_ref[...] - mean * sc
    return sc, sh


def _tcc(r0, sums0, xg1, deg_col, bng, bnb):
    def body(r0_ref, sums_ref, xg1_ref, d_ref, g_ref, b_ref,
             yb_ref, h0_ref):
        sc, sh = _bn_coefs(sums_ref, g_ref, b_ref)
        h0 = r0_ref[...] * sc + sh
        dis = lax.rsqrt(d_ref[...])
        u2 = dis * xg1_ref[...]
        yb_ref[...] = jnp.stack(
            [u2[:, 0:32], u2[:, 32:64], u2[:, 64:96], u2[:, 96:128],
             h0[:, 0:32], h0[:, 32:64], h0[:, 64:96], h0[:, 96:128]], axis=0)
        h0_ref[...] = h0

    return pl.pallas_call(
        body, grid=(NP // BPR,),
        in_specs=[_rowspec(DIM), _fullspec((8, DIM)), _rowspec(DIM),
                  _rowspec(1), _fullspec((1, DIM)), _fullspec((1, DIM))],
        out_specs=[pl.BlockSpec((8, BPR, 32), lambda i: (0, i, 0)),
                   _rowspec(DIM)],
        out_shape=[jax.ShapeDtypeStruct((8, NP, 32), jnp.float32),
                   jax.ShapeDtypeStruct((NP, DIM), jnp.float32)],
    )(r0, sums0, xg1, deg_col, bng, bnb)


def _tcd(zb, xg1, h0, deg_col, w2, b2, gw1, gb1, gw2, gb2):
    def body(z0, z1, z2, z3, z4, z5, z6, z7, xg1_ref, h0_ref, d_ref,
             w2_ref, b2_ref, gw1_ref,
             gb1_ref, gw2_ref, gb2_ref, xg2_ref, r1_ref, sums_ref):
        i = pl.program_id(0)
        au2 = jnp.concatenate([z0[...], z1[...], z2[...], z3[...]], axis=1)
        dis = lax.rsqrt(d_ref[...])
        xg1_ = xg1_ref[...]
        t2 = au2 + dis * xg1_
        xg2 = jnp.maximum(
            dis * jnp.dot(t2, w2_ref[...],
                          preferred_element_type=jnp.float32) + b2_ref[...],
            0.0)
        ah0 = jnp.concatenate([z4[...], z5[...], z6[...], z7[...]], axis=1)
        g1 = ah0 + h0_ref[...]
        t = jnp.maximum(
            jnp.dot(g1, gw1_ref[...], preferred_element_type=jnp.float32)
            + gb1_ref[...], 0.0)
        r1 = jnp.maximum(
            jnp.dot(t, gw2_ref[...], preferred_element_type=jnp.float32)
            + gb2_ref[...], 0.0)
        xg2_ref[...] = xg2
        r1_ref[...] = r1
        _sums_update(i, r1, sums_ref)

    return pl.pallas_call(
        body, grid=(NP // BPR,),
        in_specs=[_zspec(0), _zspec(1), _zspec(2), _zspec(3), _zspec(4),
                  _zspec(5), _zspec(6), _zspec(7),
                  _rowspec(DIM), _rowspec(DIM), _rowspec(1),
                  _fullspec((DIM, DIM)), _fullspec((1, DIM)),
                  _fullspec((DIM, DIM)), _fullspec((1, DIM)),
                  _fullspec((DIM, DIM)), _fullspec((1, DIM))],
        out_specs=[_rowspec(DIM), _rowspec(DIM),
                   pl.BlockSpec((8, DIM), lambda i: (0, 0))],
        out_shape=[jax.ShapeDtypeStruct((NP, DIM), jnp.float32),
                   jax.ShapeDtypeStruct((NP, DIM), jnp.float32),
                   jax.ShapeDtypeStruct((8, DIM), jnp.float32)],
    )(zb, zb, zb, zb, zb, zb, zb, zb,
      xg1, h0, deg_col, w2, b2, gw1, gb1, gw2, gb2)


def _tce(r1, sums1, bng, bnb):
    def body(r1_ref, sums_ref, g_ref, b_ref, yc_ref, h1_ref):
        sc, sh = _bn_coefs(sums_ref, g_ref, b_ref)
        h1 = r1_ref[...] * sc + sh
        yc_ref[...] = jnp.stack(
            [h1[:, 0:32], h1[:, 32:64], h1[:, 64:96], h1[:, 96:128]], axis=0)
        h1_ref[...] = h1

    return pl.pallas_call(
        body, grid=(NP // BPR,),
        in_specs=[_rowspec(DIM), _fullspec((8, DIM)),
                  _fullspec((1, DIM)), _fullspec((1, DIM))],
        out_specs=[pl.BlockSpec((4, BPR, 32), lambda i: (0, i, 0)),
                   _rowspec(DIM)],
        out_shape=[jax.ShapeDtypeStruct((4, NP, 32), jnp.float32),
                   jax.ShapeDtypeStruct((NP, DIM), jnp.float32)],
    )(r1, sums1, bng, bnb)


def _tcf(zc, h1, gw1, gb1, gw2, gb2):
    def body(z0, z1, z2, z3, h1_ref, gw1_ref, gb1_ref, gw2_ref, gb2_ref,
             r2_ref, sums_ref):
        i = pl.program_id(0)
        g2 = jnp.concatenate([z0[...], z1[...], z2[...], z3[...]], axis=1) \
            + h1_ref[...]
        t = jnp.maximum(
            jnp.dot(g2, gw1_ref[...], preferred_element_type=jnp.float32)
            + gb1_ref[...], 0.0)
        r2 = jnp.maximum(
            jnp.dot(t, gw2_ref[...], preferred_element_type=jnp.float32)
            + gb2_ref[...], 0.0)
        r2_ref[...] = r2
        _sums_update(i, r2, sums_ref)

    return pl.pallas_call(
        body, grid=(NP // BPR,),
        in_specs=[_zspec(0), _zspec(1), _zspec(2), _zspec(3),
                  _rowspec(DIM),
                  _fullspec((DIM, DIM)), _fullspec((1, DIM)),
                  _fullspec((DIM, DIM)), _fullspec((1, DIM))],
        out_specs=[_rowspec(DIM), pl.BlockSpec((8, DIM), lambda i: (0, 0))],
        out_shape=[jax.ShapeDtypeStruct((NP, DIM), jnp.float32),
                   jax.ShapeDtypeStruct((8, DIM), jnp.float32)],
    )(zc, zc, zc, zc, h1, gw1, gb1, gw2, gb2)


def _tcg(r2, sums2, bng, bnb):
    def body(r2_ref, sums_ref, g_ref, b_ref, h2_ref):
        sc, sh = _bn_coefs(sums_ref, g_ref, b_ref)
        h2_ref[...] = r2_ref[...] * sc + sh

    return pl.pallas_call(
        body, grid=(NP // BPR,),
        in_specs=[_rowspec(DIM), _fullspec((8, DIM)),
                  _fullspec((1, DIM)), _fullspec((1, DIM))],
        out_specs=_rowspec(DIM),
        out_shape=jax.ShapeDtypeStruct((NP, DIM), jnp.float32),
    )(r2, sums2, bng, bnb)


def kernel(x, edge_index, batch, W_gcn1, b_gcn1, W_gcn2, b_gcn2,
           g0_W1, g0_b1, g0_W2, g0_b2, bn0_g, bn0_b,
           g1_W1, g1_b1, g1_W2, g1_b2, bn1_g, bn1_b,
           g2_W1, g2_b1, g2_W2, g2_b2, bn2_g, bn2_b):
    row = lambda v: v.reshape(1, DIM)
    padw = lambda w: jnp.pad(w, ((0, 96 - w.shape[0]), (0, 0)))

    xp = jnp.pad(x, ((0, NP - N), (0, 96 - 78)))
    batch_pad = jnp.pad(batch, (0, NP - N), constant_values=NG - 1)
    ei_pad = jnp.pad(edge_index, ((0, 0), (0, E_PAD - E)),
                     constant_values=NP - 1)
    src_flat = ei_pad[0]
    dst2d = ei_pad[1].reshape(16 * NBE, 128)

    deg, starts = _sc_prep(ei_pad, batch_pad)
    deg_col = deg.reshape(NP, 1)

    ya = _tca(xp, deg_col)
    za = _spmm6(ya.reshape(6 * NP, 32), src_flat, dst2d)
    xg1, r0, sums0 = _tcb(za, xp, deg_col, padw(W_gcn1), row(b_gcn1),
                          padw(g0_W1), row(g0_b1), g0_W2, row(g0_b2))
    yb, h0 = _tcc(r0, sums0, xg1, deg_col, row(bn0_g), row(bn0_b))
    zb = _spmm8(yb.reshape(8 * NP, 32), src_flat, dst2d)
    xg2, r1, sums1 = _tcd(zb, xg1, h0, deg_col, W_gcn2, row(b_gcn2),
                          g1_W1, row(g1_b1), g1_W2, row(g1_b2))
    yc, h1 = _tce(r1, sums1, row(bn1_g), row(bn1_b))
    zc = _spmm4(yc.reshape(4 * NP, 32), src_flat, dst2d)
    r2, sums2 = _tcf(zc, h1, g2_W1, row(g2_b1), g2_W2, row(g2_b2))
    h2 = _tcg(r2, sums2, row(bn2_g), row(bn2_b))
    out = _sc_segmax(h0, h1, h2, xg1, xg2, starts)
    return out.reshape(NG, FREL)
